# Initial kernel scaffold; baseline (speedup 1.0000x reference)
import jax
import jax.numpy as jnp
from jax.experimental import pallas as pl


def kernel(x, edge_index, edge_attr, batch, embed, W1, b1, W2, b2):
    raise NotImplementedError("write your pallas kernel here")


# speedup vs baseline: 15.7001x; 15.7001x over previous
import functools

import jax
import jax.numpy as jnp
from jax import lax
from jax.experimental import pallas as pl
from jax.experimental.pallas import tpu as pltpu
from jax.experimental.pallas import tpu_sc as plsc

NC = 2
NS = 16
NW = NC * NS
L = 16

F = 64
FH = F // 2


def _mesh():
    return plsc.VectorSubcoreMesh(core_axis_name="c", subcore_axis_name="s")


def _sc_params():
    return pltpu.CompilerParams(
        needs_layout_passes=False, use_tc_tiling_on_sc=False)


def _vbcast(v, l):
    idx = jnp.full((L,), l, dtype=jnp.int32)
    dnums = lax.GatherDimensionNumbers(
        offset_dims=(), collapsed_slice_dims=(0,), start_index_map=(0,))
    return lax.gather(v, idx[:, None], dnums, (1,),
                      mode=lax.GatherScatterMode.PROMISE_IN_BOUNDS)


def _make_gather_deg(Np, Ep):
    n_w = Np // NW
    GCH = 112
    n_ch = n_w // GCH
    e_w = Ep // NW
    ECH = 1792
    e_ch = e_w // ECH

    @functools.partial(
        pl.kernel,
        out_type=(
            jax.ShapeDtypeStruct((2 * Np, FH), jnp.float32),
            jax.ShapeDtypeStruct((NW, Np), jnp.float32),
        ),
        mesh=_mesh(),
        compiler_params=_sc_params(),
        scratch_types=[
            pltpu.VMEM((n_w,), jnp.int32),
            pltpu.VMEM((GCH,), jnp.int32),
            pltpu.VMEM((GCH, FH), jnp.float32),
            pltpu.VMEM((Np,), jnp.float32),
            pltpu.VMEM((ECH,), jnp.int32),
            pltpu.VMEM((ECH,), jnp.float32),
            pltpu.SemaphoreType.DMA,
        ],
    )
    def k(embed2_h, xp_h, dstf_h, ewf_h, h0_h, degp_h,
          xb, idxb, rowsb, degb, dstb, ewb, sem):
        c = lax.axis_index("c")
        s = lax.axis_index("s")
        wid = s * NC + c
        nbase = wid * n_w

        pltpu.sync_copy(xp_h.at[pl.ds(nbase, n_w)], xb)

        def gather_half(off, out_base):
            def chunk(ch, _):
                def fill(i, _):
                    v = xb[pl.ds(ch * GCH + i * L, L)]
                    idxb[pl.ds(i * L, L)] = v * 2 + off
                    return 0
                lax.fori_loop(0, GCH // L, fill, 0)
                pltpu.async_copy(embed2_h.at[idxb], rowsb, sem).wait()
                pltpu.sync_copy(
                    rowsb, h0_h.at[pl.ds(out_base + ch * GCH, GCH)])
                return 0
            lax.fori_loop(0, n_ch, chunk, 0)

        gather_half(0, nbase)
        gather_half(1, Np + nbase)

        def zero(i, _):
            degb[pl.ds(i * L, L)] = jnp.zeros((L,), jnp.float32)
            return 0
        lax.fori_loop(0, Np // L, zero, 0)

        ebase = wid * e_w

        def echunk(ch, _):
            pltpu.sync_copy(dstf_h.at[pl.ds(ebase + ch * ECH, ECH)], dstb)
            pltpu.sync_copy(ewf_h.at[pl.ds(ebase + ch * ECH, ECH)], ewb)

            def grp(i, _):
                di = dstb[pl.ds(i * L, L)]
                wv = ewb[pl.ds(i * L, L)]
                plsc.addupdate_scatter(degb, [di], wv)
                return 0
            lax.fori_loop(0, ECH // L, grp, 0)
            return 0
        lax.fori_loop(0, e_ch, echunk, 0)

        pltpu.sync_copy(degb, degp_h.at[wid])

    return k


def _make_dinv(Np):
    R = Np // 128
    BR = 8
    grid = R // BR

    def body(degp_ref, dinv_ref, d2_ref):
        deg = jnp.sum(degp_ref[...], axis=0) + 1.0
        dinv = jnp.where(deg > 0, lax.rsqrt(deg), 0.0)
        dinv_ref[...] = dinv
        d2_ref[...] = dinv * dinv

    return pl.pallas_call(
        body,
        grid=(grid,),
        in_specs=[pl.BlockSpec((NW, BR, 128), lambda i: (0, i, 0))],
        out_specs=[pl.BlockSpec((BR, 128), lambda i: (i, 0)),
                   pl.BlockSpec((BR, 128), lambda i: (i, 0))],
        out_shape=[jax.ShapeDtypeStruct((R, 128), jnp.float32),
                   jax.ShapeDtypeStruct((R, 128), jnp.float32)],
    )


def _make_prep(Np, BR=1024):
    grid = Np // BR

    def body(h_ref, dinv_ref, d2_ref, g_ref, hs_ref):
        dv = dinv_ref[...]
        d2 = d2_ref[...]
        g_ref[0] = h_ref[0] * dv
        g_ref[1] = h_ref[1] * dv
        hs_ref[0] = h_ref[0] * d2
        hs_ref[1] = h_ref[1] * d2

    return pl.pallas_call(
        body,
        grid=(grid,),
        in_specs=[pl.BlockSpec((2, BR, FH), lambda i: (0, i, 0)),
                  pl.BlockSpec((BR, 1), lambda i: (i, 0)),
                  pl.BlockSpec((BR, 1), lambda i: (i, 0))],
        out_specs=[pl.BlockSpec((2, BR, FH), lambda i: (0, i, 0)),
                   pl.BlockSpec((2, BR, FH), lambda i: (0, i, 0))],
        out_shape=[jax.ShapeDtypeStruct((2, Np, FH), jnp.float32),
                   jax.ShapeDtypeStruct((2, Np, FH), jnp.float32)],
    )


def _make_conv(Np, Ep):
    e_t = Ep // NS
    ECH = 512
    n_sub = ECH // 128
    n_ch = e_t // ECH
    r_t = Np // NS
    ZR = 196

    @functools.partial(
        pl.kernel,
        out_type=jax.ShapeDtypeStruct((2 * Np, FH), jnp.float32),
        mesh=_mesh(),
        compiler_params=_sc_params(),
        scratch_types=[
            pltpu.VMEM((ECH,), jnp.int32),
            pltpu.VMEM((ECH,), jnp.float32),
            pltpu.VMEM((n_sub, 128), jnp.int32),
            pltpu.VMEM((n_sub, 128), jnp.int32),
            pltpu.VMEM((ECH, FH), jnp.float32),
            pltpu.VMEM((ZR, FH), jnp.float32),
            pltpu.VMEM_SHARED((Np, FH), jnp.float32),
            pltpu.SemaphoreType.DMA,
            pltpu.SemaphoreType.DMA,
        ],
    )
    def k(g_h, srcf_h, dst2d_h, ewf_h, out_h,
          srcb, ewb, gidx, didx, rowsb, zbuf, acc, sem, sem2):
        c = lax.axis_index("c")
        s = lax.axis_index("s")
        coff = c * Np

        def zf(i, _):
            zbuf[i, pl.ds(0, L)] = jnp.zeros((L,), jnp.float32)
            zbuf[i, pl.ds(L, L)] = jnp.zeros((L,), jnp.float32)
            return 0
        lax.fori_loop(0, ZR, zf, 0)

        def zcp(i, _):
            pltpu.sync_copy(zbuf, acc.at[pl.ds(s * r_t + i * ZR, ZR)])
            return 0
        lax.fori_loop(0, r_t // ZR, zcp, 0)
        plsc.subcore_barrier()

        ebase = s * e_t

        def chunk(ch, _):
            base = ebase + ch * ECH
            pltpu.sync_copy(srcf_h.at[pl.ds(base, ECH)], srcb)
            pltpu.sync_copy(ewf_h.at[pl.ds(base, ECH)], ewb)
            pltpu.sync_copy(dst2d_h.at[pl.ds(base // 128, n_sub)], didx)

            def fill(i, _):
                sv = srcb[pl.ds(i * L, L)]
                j = i // 8
                lo = (i % 8) * L
                gidx[j, pl.ds(lo, L)] = sv + coff
                return 0
            lax.fori_loop(0, ECH // L, fill, 0)

            descs = [pltpu.async_copy(g_h.at[gidx.at[j]],
                                      rowsb.at[pl.ds(j * 128, 128)], sem)
                     for j in range(n_sub)]
            for d in descs:
                d.wait()

            def grp(i, _):
                wv = ewb[pl.ds(i * L, L)]
                for l in range(L):
                    nb = _vbcast(wv, l)
                    r = i * L + l
                    rowsb[r, pl.ds(0, L)] = rowsb[r, pl.ds(0, L)] * nb
                    rowsb[r, pl.ds(L, L)] = rowsb[r, pl.ds(L, L)] * nb
                return 0
            lax.fori_loop(0, ECH // L, grp, 0)

            descs2 = [pltpu.async_copy(rowsb.at[pl.ds(j * 128, 128)],
                                       acc.at[didx.at[j]], sem2, add=True)
                      for j in range(n_sub)]
            for d in descs2:
                d.wait()
            return 0

        lax.fori_loop(0, n_ch, chunk, 0)

        plsc.subcore_barrier()
        pltpu.sync_copy(acc.at[pl.ds(s * r_t, r_t)],
                        out_h.at[pl.ds(coff + s * r_t, r_t)])

    return k


def _make_mlp(Np, BR=1024):
    grid = Np // BR

    def body(acc_ref, hs_ref, dinv_ref, d2_ref, w1_ref, b1_ref, w2_ref,
             b2_ref, g2_ref, hs2_ref):
        dv = dinv_ref[...]
        d2 = d2_ref[...]
        x = jnp.concatenate([acc_ref[0], acc_ref[1]], axis=1) * dv
        x = x + jnp.concatenate([hs_ref[0], hs_ref[1]], axis=1)
        h = jnp.dot(x, w1_ref[...], precision=lax.Precision.HIGHEST,
                    preferred_element_type=jnp.float32) + b1_ref[...]
        h = jnp.maximum(h, 0.0)
        h2 = jnp.dot(h, w2_ref[...], precision=lax.Precision.HIGHEST,
                     preferred_element_type=jnp.float32)
        g2 = h2 * dv
        hs2 = h2 * d2 + b2_ref[...]
        g2_ref[0] = g2[:, :FH]
        g2_ref[1] = g2[:, FH:]
        hs2_ref[0] = hs2[:, :FH]
        hs2_ref[1] = hs2[:, FH:]

    return pl.pallas_call(
        body,
        grid=(grid,),
        in_specs=[pl.BlockSpec((2, BR, FH), lambda i: (0, i, 0)),
                  pl.BlockSpec((2, BR, FH), lambda i: (0, i, 0)),
                  pl.BlockSpec((BR, 1), lambda i: (i, 0)),
                  pl.BlockSpec((BR, 1), lambda i: (i, 0)),
                  pl.BlockSpec((F, 2 * F), lambda i: (0, 0)),
                  pl.BlockSpec((1, 2 * F), lambda i: (0, 0)),
                  pl.BlockSpec((2 * F, F), lambda i: (0, 0)),
                  pl.BlockSpec((1, F), lambda i: (0, 0))],
        out_specs=[pl.BlockSpec((2, BR, FH), lambda i: (0, i, 0)),
                   pl.BlockSpec((2, BR, FH), lambda i: (0, i, 0))],
        out_shape=[jax.ShapeDtypeStruct((2, Np, FH), jnp.float32),
                   jax.ShapeDtypeStruct((2, Np, FH), jnp.float32)],
    )


def _make_fma(Np, BR=1024):
    grid = Np // BR

    def body(acc_ref, hs_ref, dinv_ref, p_ref):
        dv = dinv_ref[...]
        p_ref[0] = acc_ref[0] * dv + hs_ref[0]
        p_ref[1] = acc_ref[1] * dv + hs_ref[1]

    return pl.pallas_call(
        body,
        grid=(grid,),
        in_specs=[pl.BlockSpec((2, BR, FH), lambda i: (0, i, 0)),
                  pl.BlockSpec((2, BR, FH), lambda i: (0, i, 0)),
                  pl.BlockSpec((BR, 1), lambda i: (i, 0))],
        out_specs=[pl.BlockSpec((2, BR, FH), lambda i: (0, i, 0))],
        out_shape=[jax.ShapeDtypeStruct((2, Np, FH), jnp.float32)],
    )


def _make_segmax(Np, G):
    r_t = Np // NS
    TG = G + 8

    @functools.partial(
        pl.kernel,
        out_type=jax.ShapeDtypeStruct((2 * G, FH), jnp.float32),
        mesh=_mesh(),
        compiler_params=_sc_params(),
        scratch_types=[
            pltpu.VMEM((r_t, FH), jnp.float32),
            pltpu.VMEM((r_t,), jnp.int32),
            pltpu.VMEM((TG, FH), jnp.float32),
            pltpu.VMEM((8, FH), jnp.float32),
            pltpu.VMEM((8, FH), jnp.float32),
            pltpu.VMEM_SHARED((NS, G, FH), jnp.float32),
            pltpu.SemaphoreType.DMA,
        ],
    )
    def k(p_h, batch_h, out_h, rowsb, batchb, tb, racc, rtmp, shared, sem):
        c = lax.axis_index("c")
        s = lax.axis_index("s")
        coff = c * Np

        pltpu.sync_copy(p_h.at[pl.ds(coff + s * r_t, r_t)], rowsb)
        pltpu.sync_copy(batch_h.at[pl.ds(s * r_t, r_t)], batchb)

        ninf = jnp.full((L,), -jnp.inf, jnp.float32)

        def zt(i, _):
            tb[i, pl.ds(0, L)] = ninf
            tb[i, pl.ds(L, L)] = ninf
            return 0
        lax.fori_loop(0, TG, zt, 0)

        iota = lax.iota(jnp.int32, L)

        def grp(i, _):
            bv = batchb[pl.ds(i * L, L)]
            for l in range(L):
                gb = _vbcast(bv, l)
                r = i * L + l
                r0 = rowsb[r, pl.ds(0, L)]
                r1 = rowsb[r, pl.ds(L, L)]
                cur0 = plsc.load_gather(tb, [gb, iota])
                cur1 = plsc.load_gather(tb, [gb, iota + L])
                plsc.store_scatter(tb, [gb, iota], jnp.maximum(cur0, r0))
                plsc.store_scatter(tb, [gb, iota + L], jnp.maximum(cur1, r1))
            return 0
        lax.fori_loop(0, r_t // L, grp, 0)

        pltpu.sync_copy(tb.at[pl.ds(0, G)], shared.at[s])
        plsc.subcore_barrier()

        gs = G // NS
        pltpu.sync_copy(shared.at[0, pl.ds(s * gs, gs)], racc)

        for t in range(1, NS):
            pltpu.sync_copy(shared.at[t, pl.ds(s * gs, gs)], rtmp)
            for r in range(gs):
                for j in (0, L):
                    racc[r, pl.ds(j, L)] = jnp.maximum(
                        racc[r, pl.ds(j, L)], rtmp[r, pl.ds(j, L)])

        pltpu.sync_copy(racc, out_h.at[pl.ds(c * G + s * gs, gs)])

    return k


def kernel(x, edge_index, edge_attr, batch, embed, W1, b1, W2, b2):
    N = x.shape[0]
    E = edge_index.shape[1]
    V = embed.shape[0]
    G = 128

    Np = 50176
    Ep = 802816
    assert N <= Np and E <= Ep

    pad_n = Np - N
    pad_e = Ep - E

    xp = jnp.concatenate([x, (jnp.arange(pad_n, dtype=jnp.int32) % V)])
    srcf = jnp.concatenate(
        [edge_index[0], (jnp.arange(pad_e, dtype=jnp.int32) * 131) % N])
    dstf = jnp.concatenate(
        [edge_index[1], (jnp.arange(pad_e, dtype=jnp.int32) * 137) % N])
    ewf = jnp.concatenate([edge_attr, jnp.zeros((pad_e,), jnp.float32)])
    batchp = jnp.concatenate([batch, jnp.full((pad_n,), G, jnp.int32)])
    dst2d = dstf.reshape(Ep // 128, 128)

    embed2 = embed.reshape(V * 2, FH)

    h0_st, degp = _make_gather_deg(Np, Ep)(embed2, xp, dstf, ewf)

    dinv2d, d2_2d = _make_dinv(Np)(degp.reshape(NW, Np // 128, 128))
    dinvcol = dinv2d.reshape(Np, 1)
    d2col = d2_2d.reshape(Np, 1)

    g0_3, hs0_3 = _make_prep(Np)(h0_st.reshape(2, Np, FH), dinvcol, d2col)

    conv = _make_conv(Np, Ep)

    acc0_st = conv(g0_3.reshape(2 * Np, FH), srcf, dst2d, ewf)

    g2_3, hs2_3 = _make_mlp(Np)(acc0_st.reshape(2, Np, FH), hs0_3,
                                dinvcol, d2col,
                                W1, b1.reshape(1, 2 * F), W2,
                                b2.reshape(1, F))

    acc2_st = conv(g2_3.reshape(2 * Np, FH), srcf, dst2d, ewf)

    (p2_3,) = _make_fma(Np)(acc2_st.reshape(2, Np, FH), hs2_3, dinvcol)

    out_st = _make_segmax(Np, G)(p2_3.reshape(2 * Np, FH), batchp)

    out3 = out_st.reshape(2, G, FH)
    return jnp.concatenate([out3[0], out3[1]], axis=1)


# speedup vs baseline: 19.5600x; 1.2459x over previous
import functools

import jax
import jax.numpy as jnp
from jax import lax
from jax.experimental import pallas as pl
from jax.experimental.pallas import tpu as pltpu
from jax.experimental.pallas import tpu_sc as plsc

NC = 2
NS = 16
NW = NC * NS
L = 16

F = 64
FH = F // 2


def _mesh():
    return plsc.VectorSubcoreMesh(core_axis_name="c", subcore_axis_name="s")


def _sc_params():
    return pltpu.CompilerParams(
        needs_layout_passes=False, use_tc_tiling_on_sc=False)


def _vbcast(v, l):
    idx = jnp.full((L,), l, dtype=jnp.int32)
    dnums = lax.GatherDimensionNumbers(
        offset_dims=(), collapsed_slice_dims=(0,), start_index_map=(0,))
    return lax.gather(v, idx[:, None], dnums, (1,),
                      mode=lax.GatherScatterMode.PROMISE_IN_BOUNDS)


def _make_gather_deg(Np, Ep):
    n_w = Np // NW
    GCH = 112
    n_ch = n_w // GCH
    e_w = Ep // NW
    ECH = 1792
    e_ch = e_w // ECH

    @functools.partial(
        pl.kernel,
        out_type=(
            jax.ShapeDtypeStruct((2 * Np, FH), jnp.float32),
            jax.ShapeDtypeStruct((NW, Np), jnp.float32),
        ),
        mesh=_mesh(),
        compiler_params=_sc_params(),
        scratch_types=[
            pltpu.VMEM((n_w,), jnp.int32),
            pltpu.VMEM((GCH,), jnp.int32),
            pltpu.VMEM((GCH, FH), jnp.float32),
            pltpu.VMEM((Np,), jnp.float32),
            pltpu.VMEM((ECH,), jnp.int32),
            pltpu.VMEM((ECH,), jnp.float32),
            pltpu.SemaphoreType.DMA,
        ],
    )
    def k(embed2_h, xp_h, dstf_h, ewf_h, h0_h, degp_h,
          xb, idxb, rowsb, degb, dstb, ewb, sem):
        c = lax.axis_index("c")
        s = lax.axis_index("s")
        wid = s * NC + c
        nbase = wid * n_w

        pltpu.sync_copy(xp_h.at[pl.ds(nbase, n_w)], xb)

        def gather_half(off, out_base):
            def chunk(ch, _):
                def fill(i, _):
                    v = xb[pl.ds(ch * GCH + i * L, L)]
                    idxb[pl.ds(i * L, L)] = v * 2 + off
                    return 0
                lax.fori_loop(0, GCH // L, fill, 0)
                pltpu.async_copy(embed2_h.at[idxb], rowsb, sem).wait()
                pltpu.sync_copy(
                    rowsb, h0_h.at[pl.ds(out_base + ch * GCH, GCH)])
                return 0
            lax.fori_loop(0, n_ch, chunk, 0)

        gather_half(0, nbase)
        gather_half(1, Np + nbase)

        def zero(i, _):
            degb[pl.ds(i * L, L)] = jnp.zeros((L,), jnp.float32)
            return 0
        lax.fori_loop(0, Np // L, zero, 0)

        ebase = wid * e_w

        def echunk(ch, _):
            pltpu.sync_copy(dstf_h.at[pl.ds(ebase + ch * ECH, ECH)], dstb)
            pltpu.sync_copy(ewf_h.at[pl.ds(ebase + ch * ECH, ECH)], ewb)

            def grp(i, _):
                di = dstb[pl.ds(i * L, L)]
                wv = ewb[pl.ds(i * L, L)]
                plsc.addupdate_scatter(degb, [di], wv)
                return 0
            lax.fori_loop(0, ECH // L, grp, 0)
            return 0
        lax.fori_loop(0, e_ch, echunk, 0)

        pltpu.sync_copy(degb, degp_h.at[wid])

    return k


def _make_dinv(Np):
    R = Np // 128
    BR = 8
    grid = R // BR

    def body(degp_ref, dinv_ref, d2_ref):
        deg = jnp.sum(degp_ref[...], axis=0) + 1.0
        dinv = jnp.where(deg > 0, lax.rsqrt(deg), 0.0)
        dinv_ref[...] = dinv
        d2_ref[...] = dinv * dinv

    return pl.pallas_call(
        body,
        grid=(grid,),
        in_specs=[pl.BlockSpec((NW, BR, 128), lambda i: (0, i, 0))],
        out_specs=[pl.BlockSpec((BR, 128), lambda i: (i, 0)),
                   pl.BlockSpec((BR, 128), lambda i: (i, 0))],
        out_shape=[jax.ShapeDtypeStruct((R, 128), jnp.float32),
                   jax.ShapeDtypeStruct((R, 128), jnp.float32)],
    )


def _make_prep(Np, BR=1024):
    grid = Np // BR

    def body(h_ref, dinv_ref, d2_ref, g_ref, hs_ref):
        dv = dinv_ref[...]
        d2 = d2_ref[...]
        g_ref[0] = h_ref[0] * dv
        g_ref[1] = h_ref[1] * dv
        hs_ref[0] = h_ref[0] * d2
        hs_ref[1] = h_ref[1] * d2

    return pl.pallas_call(
        body,
        grid=(grid,),
        in_specs=[pl.BlockSpec((2, BR, FH), lambda i: (0, i, 0)),
                  pl.BlockSpec((BR, 1), lambda i: (i, 0)),
                  pl.BlockSpec((BR, 1), lambda i: (i, 0))],
        out_specs=[pl.BlockSpec((2, BR, FH), lambda i: (0, i, 0)),
                   pl.BlockSpec((2, BR, FH), lambda i: (0, i, 0))],
        out_shape=[jax.ShapeDtypeStruct((2, Np, FH), jnp.float32),
                   jax.ShapeDtypeStruct((2, Np, FH), jnp.float32)],
    )


def _make_conv(Np, Ep):
    e_t = Ep // NS
    ECH = 256
    n_sub = ECH // 128
    n_ch = e_t // ECH
    r_t = Np // NS
    ZR = 196

    @functools.partial(
        pl.kernel,
        out_type=jax.ShapeDtypeStruct((2 * Np, FH), jnp.float32),
        mesh=_mesh(),
        compiler_params=_sc_params(),
        scratch_types=[
            pltpu.VMEM((2, ECH), jnp.int32),
            pltpu.VMEM((2, ECH), jnp.float32),
            pltpu.VMEM((2 * n_sub, 128), jnp.int32),
            pltpu.VMEM((2 * n_sub, 128), jnp.int32),
            pltpu.VMEM((2 * ECH, FH), jnp.float32),
            pltpu.VMEM((ZR, FH), jnp.float32),
            pltpu.VMEM_SHARED((Np, FH), jnp.float32),
            pltpu.SemaphoreType.DMA,
            pltpu.SemaphoreType.DMA,
            pltpu.SemaphoreType.DMA,
        ],
    )
    def k(g_h, srcf_h, dst2d_h, ewf_h, out_h,
          srcb, ewb, gidx, didx, rowsb, zbuf, acc, sem, sem2, sem3):
        c = lax.axis_index("c")
        s = lax.axis_index("s")
        coff = c * Np

        def zf(i, _):
            zbuf[i, pl.ds(0, L)] = jnp.zeros((L,), jnp.float32)
            zbuf[i, pl.ds(L, L)] = jnp.zeros((L,), jnp.float32)
            return 0
        lax.fori_loop(0, ZR, zf, 0)

        def zcp(i, _):
            pltpu.sync_copy(zbuf, acc.at[pl.ds(s * r_t + i * ZR, ZR)])
            return 0
        lax.fori_loop(0, r_t // ZR, zcp, 0)
        plsc.subcore_barrier()

        ebase = s * e_t

        def load_idx(ch, ph):
            base = ebase + ch * ECH
            return [pltpu.async_copy(srcf_h.at[pl.ds(base, ECH)],
                                     srcb.at[ph], sem3),
                    pltpu.async_copy(ewf_h.at[pl.ds(base, ECH)],
                                     ewb.at[ph], sem3),
                    pltpu.async_copy(dst2d_h.at[pl.ds(base // 128, n_sub)],
                                     didx.at[pl.ds(ph * n_sub, n_sub)], sem3)]

        def wait_idx(ch, ph):
            base = ebase + ch * ECH
            pltpu.make_async_copy(srcf_h.at[pl.ds(base, ECH)],
                                  srcb.at[ph], sem3).wait()
            pltpu.make_async_copy(ewf_h.at[pl.ds(base, ECH)],
                                  ewb.at[ph], sem3).wait()
            pltpu.make_async_copy(dst2d_h.at[pl.ds(base // 128, n_sub)],
                                  didx.at[pl.ds(ph * n_sub, n_sub)],
                                  sem3).wait()

        def fire_gather(ph):
            def fill(i, _):
                sv = srcb[ph, pl.ds(i * L, L)]
                j = i // 8
                lo = (i % 8) * L
                gidx[ph * n_sub + j, pl.ds(lo, L)] = sv + coff
                return 0
            lax.fori_loop(0, ECH // L, fill, 0)
            for j in range(n_sub):
                pltpu.async_copy(
                    g_h.at[gidx.at[ph * n_sub + j]],
                    rowsb.at[pl.ds((ph * n_sub + j) * 128, 128)], sem)

        def wait_gather(ph):
            for j in range(n_sub):
                pltpu.make_async_copy(
                    g_h.at[gidx.at[ph * n_sub + j]],
                    rowsb.at[pl.ds((ph * n_sub + j) * 128, 128)], sem).wait()

        def scale(ph):
            def grp(i, _):
                wv = ewb[ph, pl.ds(i * L, L)]
                for l in range(L):
                    nb = _vbcast(wv, l)
                    r = ph * ECH + i * L + l
                    rowsb[r, pl.ds(0, L)] = rowsb[r, pl.ds(0, L)] * nb
                    rowsb[r, pl.ds(L, L)] = rowsb[r, pl.ds(L, L)] * nb
                return 0
            lax.fori_loop(0, ECH // L, grp, 0)

        def fire_scatter(ph):
            for j in range(n_sub):
                pltpu.async_copy(
                    rowsb.at[pl.ds((ph * n_sub + j) * 128, 128)],
                    acc.at[didx.at[ph * n_sub + j]], sem2, add=True)

        def wait_scatter(ph):
            for j in range(n_sub):
                pltpu.make_async_copy(
                    rowsb.at[pl.ds((ph * n_sub + j) * 128, 128)],
                    acc.at[didx.at[ph * n_sub + j]], sem2).wait()

        for d in load_idx(0, 0):
            d.wait()
        fire_gather(0)
        load_idx(1, 1)
        wait_gather(0)
        scale(0)
        fire_scatter(0)
        wait_idx(1, 1)
        fire_gather(1)

        def body(ch, _):
            ph = lax.rem(ch, 2)
            nxt = 1 - ph
            wait_scatter(nxt)
            load_idx(ch + 1, nxt)
            wait_gather(ph)
            scale(ph)
            fire_scatter(ph)
            wait_idx(ch + 1, nxt)
            fire_gather(nxt)
            return 0
        lax.fori_loop(1, n_ch - 1, body, 0)

        phl = (n_ch - 1) % 2
        wait_scatter(1 - phl)
        wait_gather(phl)
        scale(phl)
        fire_scatter(phl)
        wait_scatter(phl)

        plsc.subcore_barrier()
        pltpu.sync_copy(acc.at[pl.ds(s * r_t, r_t)],
                        out_h.at[pl.ds(coff + s * r_t, r_t)])

    return k


def _make_mlp(Np, BR=1024):
    grid = Np // BR

    def body(acc_ref, hs_ref, dinv_ref, d2_ref, w1_ref, b1_ref, w2_ref,
             b2_ref, g2_ref, hs2_ref):
        dv = dinv_ref[...]
        d2 = d2_ref[...]
        x = jnp.concatenate([acc_ref[0], acc_ref[1]], axis=1) * dv
        x = x + jnp.concatenate([hs_ref[0], hs_ref[1]], axis=1)
        h = jnp.dot(x, w1_ref[...], precision=lax.Precision.HIGHEST,
                    preferred_element_type=jnp.float32) + b1_ref[...]
        h = jnp.maximum(h, 0.0)
        h2 = jnp.dot(h, w2_ref[...], precision=lax.Precision.HIGHEST,
                     preferred_element_type=jnp.float32)
        g2 = h2 * dv
        hs2 = h2 * d2 + b2_ref[...]
        g2_ref[0] = g2[:, :FH]
        g2_ref[1] = g2[:, FH:]
        hs2_ref[0] = hs2[:, :FH]
        hs2_ref[1] = hs2[:, FH:]

    return pl.pallas_call(
        body,
        grid=(grid,),
        in_specs=[pl.BlockSpec((2, BR, FH), lambda i: (0, i, 0)),
                  pl.BlockSpec((2, BR, FH), lambda i: (0, i, 0)),
                  pl.BlockSpec((BR, 1), lambda i: (i, 0)),
                  pl.BlockSpec((BR, 1), lambda i: (i, 0)),
                  pl.BlockSpec((F, 2 * F), lambda i: (0, 0)),
                  pl.BlockSpec((1, 2 * F), lambda i: (0, 0)),
                  pl.BlockSpec((2 * F, F), lambda i: (0, 0)),
                  pl.BlockSpec((1, F), lambda i: (0, 0))],
        out_specs=[pl.BlockSpec((2, BR, FH), lambda i: (0, i, 0)),
                   pl.BlockSpec((2, BR, FH), lambda i: (0, i, 0))],
        out_shape=[jax.ShapeDtypeStruct((2, Np, FH), jnp.float32),
                   jax.ShapeDtypeStruct((2, Np, FH), jnp.float32)],
    )


def _make_fma(Np, BR=1024):
    grid = Np // BR

    def body(acc_ref, hs_ref, dinv_ref, p_ref):
        dv = dinv_ref[...]
        p_ref[0] = acc_ref[0] * dv + hs_ref[0]
        p_ref[1] = acc_ref[1] * dv + hs_ref[1]

    return pl.pallas_call(
        body,
        grid=(grid,),
        in_specs=[pl.BlockSpec((2, BR, FH), lambda i: (0, i, 0)),
                  pl.BlockSpec((2, BR, FH), lambda i: (0, i, 0)),
                  pl.BlockSpec((BR, 1), lambda i: (i, 0))],
        out_specs=[pl.BlockSpec((2, BR, FH), lambda i: (0, i, 0))],
        out_shape=[jax.ShapeDtypeStruct((2, Np, FH), jnp.float32)],
    )


def _make_segmax(Np, G):
    r_t = Np // NS
    TG = G + 8

    @functools.partial(
        pl.kernel,
        out_type=jax.ShapeDtypeStruct((2 * G, FH), jnp.float32),
        mesh=_mesh(),
        compiler_params=_sc_params(),
        scratch_types=[
            pltpu.VMEM((r_t, FH), jnp.float32),
            pltpu.VMEM((r_t,), jnp.int32),
            pltpu.VMEM((TG, FH), jnp.float32),
            pltpu.VMEM((8, FH), jnp.float32),
            pltpu.VMEM((8, FH), jnp.float32),
            pltpu.VMEM_SHARED((NS, G, FH), jnp.float32),
            pltpu.SemaphoreType.DMA,
        ],
    )
    def k(p_h, batch_h, out_h, rowsb, batchb, tb, racc, rtmp, shared, sem):
        c = lax.axis_index("c")
        s = lax.axis_index("s")
        coff = c * Np

        pltpu.sync_copy(p_h.at[pl.ds(coff + s * r_t, r_t)], rowsb)
        pltpu.sync_copy(batch_h.at[pl.ds(s * r_t, r_t)], batchb)

        ninf = jnp.full((L,), -jnp.inf, jnp.float32)

        def zt(i, _):
            tb[i, pl.ds(0, L)] = ninf
            tb[i, pl.ds(L, L)] = ninf
            return 0
        lax.fori_loop(0, TG, zt, 0)

        iota = lax.iota(jnp.int32, L)

        def grp(i, _):
            bv = batchb[pl.ds(i * L, L)]
            for l in range(L):
                gb = _vbcast(bv, l)
                r = i * L + l
                r0 = rowsb[r, pl.ds(0, L)]
                r1 = rowsb[r, pl.ds(L, L)]
                cur0 = plsc.load_gather(tb, [gb, iota])
                cur1 = plsc.load_gather(tb, [gb, iota + L])
                plsc.store_scatter(tb, [gb, iota], jnp.maximum(cur0, r0))
                plsc.store_scatter(tb, [gb, iota + L], jnp.maximum(cur1, r1))
            return 0
        lax.fori_loop(0, r_t // L, grp, 0)

        pltpu.sync_copy(tb.at[pl.ds(0, G)], shared.at[s])
        plsc.subcore_barrier()

        gs = G // NS
        pltpu.sync_copy(shared.at[0, pl.ds(s * gs, gs)], racc)

        for t in range(1, NS):
            pltpu.sync_copy(shared.at[t, pl.ds(s * gs, gs)], rtmp)
            for r in range(gs):
                for j in (0, L):
                    racc[r, pl.ds(j, L)] = jnp.maximum(
                        racc[r, pl.ds(j, L)], rtmp[r, pl.ds(j, L)])

        pltpu.sync_copy(racc, out_h.at[pl.ds(c * G + s * gs, gs)])

    return k


def kernel(x, edge_index, edge_attr, batch, embed, W1, b1, W2, b2):
    N = x.shape[0]
    E = edge_index.shape[1]
    V = embed.shape[0]
    G = 128

    Np = 50176
    Ep = 802816
    assert N <= Np and E <= Ep

    pad_n = Np - N
    pad_e = Ep - E

    xp = jnp.concatenate([x, (jnp.arange(pad_n, dtype=jnp.int32) % V)])
    srcf = jnp.concatenate(
        [edge_index[0], (jnp.arange(pad_e, dtype=jnp.int32) * 131) % N])
    dstf = jnp.concatenate(
        [edge_index[1], (jnp.arange(pad_e, dtype=jnp.int32) * 137) % N])
    ewf = jnp.concatenate([edge_attr, jnp.zeros((pad_e,), jnp.float32)])
    batchp = jnp.concatenate([batch, jnp.full((pad_n,), G, jnp.int32)])
    dst2d = dstf.reshape(Ep // 128, 128)

    embed2 = embed.reshape(V * 2, FH)

    h0_st, degp = _make_gather_deg(Np, Ep)(embed2, xp, dstf, ewf)

    dinv2d, d2_2d = _make_dinv(Np)(degp.reshape(NW, Np // 128, 128))
    dinvcol = dinv2d.reshape(Np, 1)
    d2col = d2_2d.reshape(Np, 1)

    g0_3, hs0_3 = _make_prep(Np)(h0_st.reshape(2, Np, FH), dinvcol, d2col)

    conv = _make_conv(Np, Ep)

    acc0_st = conv(g0_3.reshape(2 * Np, FH), srcf, dst2d, ewf)

    g2_3, hs2_3 = _make_mlp(Np)(acc0_st.reshape(2, Np, FH), hs0_3,
                                dinvcol, d2col,
                                W1, b1.reshape(1, 2 * F), W2,
                                b2.reshape(1, F))

    acc2_st = conv(g2_3.reshape(2 * Np, FH), srcf, dst2d, ewf)

    (p2_3,) = _make_fma(Np)(acc2_st.reshape(2, Np, FH), hs2_3, dinvcol)

    out_st = _make_segmax(Np, G)(p2_3.reshape(2 * Np, FH), batchp)

    out3 = out_st.reshape(2, G, FH)
    return jnp.concatenate([out3[0], out3[1]], axis=1)


# speedup vs baseline: 19.7866x; 1.0116x over previous
import functools

import jax
import jax.numpy as jnp
from jax import lax
from jax.experimental import pallas as pl
from jax.experimental.pallas import tpu as pltpu
from jax.experimental.pallas import tpu_sc as plsc

NC = 2
NS = 16
NW = NC * NS
L = 16

F = 64
FH = F // 2


def _mesh():
    return plsc.VectorSubcoreMesh(core_axis_name="c", subcore_axis_name="s")


def _sc_params():
    return pltpu.CompilerParams(
        needs_layout_passes=False, use_tc_tiling_on_sc=False)


def _vbcast(v, l):
    idx = jnp.full((L,), l, dtype=jnp.int32)
    dnums = lax.GatherDimensionNumbers(
        offset_dims=(), collapsed_slice_dims=(0,), start_index_map=(0,))
    return lax.gather(v, idx[:, None], dnums, (1,),
                      mode=lax.GatherScatterMode.PROMISE_IN_BOUNDS)


def _make_gather_deg(Np, Ep):
    n_w = Np // NW
    GCH = 112
    n_ch = n_w // GCH
    e_w = Ep // NW
    ECH = 1792
    e_ch = e_w // ECH

    @functools.partial(
        pl.kernel,
        out_type=(
            jax.ShapeDtypeStruct((2 * Np, FH), jnp.float32),
            jax.ShapeDtypeStruct((NW, Np), jnp.float32),
        ),
        mesh=_mesh(),
        compiler_params=_sc_params(),
        scratch_types=[
            pltpu.VMEM((n_w,), jnp.int32),
            pltpu.VMEM((GCH,), jnp.int32),
            pltpu.VMEM((GCH, FH), jnp.float32),
            pltpu.VMEM((Np,), jnp.float32),
            pltpu.VMEM((ECH,), jnp.int32),
            pltpu.VMEM((ECH,), jnp.float32),
            pltpu.SemaphoreType.DMA,
        ],
    )
    def k(embed2_h, xp_h, dstf_h, ewf_h, h0_h, degp_h,
          xb, idxb, rowsb, degb, dstb, ewb, sem):
        c = lax.axis_index("c")
        s = lax.axis_index("s")
        wid = s * NC + c
        nbase = wid * n_w

        pltpu.sync_copy(xp_h.at[pl.ds(nbase, n_w)], xb)

        def gather_half(off, out_base):
            def chunk(ch, _):
                def fill(i, _):
                    v = xb[pl.ds(ch * GCH + i * L, L)]
                    idxb[pl.ds(i * L, L)] = v * 2 + off
                    return 0
                lax.fori_loop(0, GCH // L, fill, 0)
                pltpu.async_copy(embed2_h.at[idxb], rowsb, sem).wait()
                pltpu.sync_copy(
                    rowsb, h0_h.at[pl.ds(out_base + ch * GCH, GCH)])
                return 0
            lax.fori_loop(0, n_ch, chunk, 0)

        gather_half(0, nbase)
        gather_half(1, Np + nbase)

        def zero(i, _):
            degb[pl.ds(i * L, L)] = jnp.zeros((L,), jnp.float32)
            return 0
        lax.fori_loop(0, Np // L, zero, 0)

        ebase = wid * e_w

        def echunk(ch, _):
            pltpu.sync_copy(dstf_h.at[pl.ds(ebase + ch * ECH, ECH)], dstb)
            pltpu.sync_copy(ewf_h.at[pl.ds(ebase + ch * ECH, ECH)], ewb)

            def grp(i, _):
                di = dstb[pl.ds(i * L, L)]
                wv = ewb[pl.ds(i * L, L)]
                plsc.addupdate_scatter(degb, [di], wv)
                return 0
            lax.fori_loop(0, ECH // L, grp, 0)
            return 0
        lax.fori_loop(0, e_ch, echunk, 0)

        pltpu.sync_copy(degb, degp_h.at[wid])

    return k


def _make_dinv(Np):
    R = Np // 128
    BR = 8
    grid = R // BR

    def body(degp_ref, dinv_ref, d2_ref):
        deg = jnp.sum(degp_ref[...], axis=0) + 1.0
        dinv = jnp.where(deg > 0, lax.rsqrt(deg), 0.0)
        dinv_ref[...] = dinv
        d2_ref[...] = dinv * dinv

    return pl.pallas_call(
        body,
        grid=(grid,),
        in_specs=[pl.BlockSpec((NW, BR, 128), lambda i: (0, i, 0))],
        out_specs=[pl.BlockSpec((BR, 128), lambda i: (i, 0)),
                   pl.BlockSpec((BR, 128), lambda i: (i, 0))],
        out_shape=[jax.ShapeDtypeStruct((R, 128), jnp.float32),
                   jax.ShapeDtypeStruct((R, 128), jnp.float32)],
    )


def _make_prep(Np, BR=1024):
    grid = Np // BR

    def body(h_ref, dinv_ref, d2_ref, g_ref, hs_ref):
        dv = dinv_ref[...]
        d2 = d2_ref[...]
        g_ref[0] = h_ref[0] * dv
        g_ref[1] = h_ref[1] * dv
        hs_ref[0] = h_ref[0] * d2
        hs_ref[1] = h_ref[1] * d2

    return pl.pallas_call(
        body,
        grid=(grid,),
        in_specs=[pl.BlockSpec((2, BR, FH), lambda i: (0, i, 0)),
                  pl.BlockSpec((BR, 1), lambda i: (i, 0)),
                  pl.BlockSpec((BR, 1), lambda i: (i, 0))],
        out_specs=[pl.BlockSpec((2, BR, FH), lambda i: (0, i, 0)),
                   pl.BlockSpec((2, BR, FH), lambda i: (0, i, 0))],
        out_shape=[jax.ShapeDtypeStruct((2, Np, FH), jnp.float32),
                   jax.ShapeDtypeStruct((2, Np, FH), jnp.float32)],
    )


def _make_conv(Np, Ep):
    e_t = Ep // NS
    ECH = 256
    n_sub = ECH // 128
    n_ch = e_t // ECH
    r_t = Np // NS
    ZR = 196

    @functools.partial(
        pl.kernel,
        out_type=jax.ShapeDtypeStruct((2 * Np, FH), jnp.float32),
        mesh=_mesh(),
        compiler_params=_sc_params(),
        scratch_types=[
            pltpu.VMEM((2, ECH), jnp.int32),
            pltpu.VMEM((2, ECH), jnp.float32),
            pltpu.VMEM((2 * n_sub, 128), jnp.int32),
            pltpu.VMEM((2 * n_sub, 128), jnp.int32),
            pltpu.VMEM((2 * ECH, FH), jnp.float32),
            pltpu.VMEM((ZR, FH), jnp.float32),
            pltpu.VMEM_SHARED((Np, FH), jnp.float32),
            pltpu.SemaphoreType.DMA,
            pltpu.SemaphoreType.DMA,
            pltpu.SemaphoreType.DMA,
        ],
    )
    def k(g_h, srcf_h, dst2d_h, ewf_h, dinv_h, hs_h, out_h,
          srcb, ewb, gidx, didx, rowsb, zbuf, acc, sem, sem2, sem3):
        c = lax.axis_index("c")
        s = lax.axis_index("s")
        coff = c * Np

        def zf(i, _):
            zbuf[i, pl.ds(0, L)] = jnp.zeros((L,), jnp.float32)
            zbuf[i, pl.ds(L, L)] = jnp.zeros((L,), jnp.float32)
            return 0
        lax.fori_loop(0, ZR, zf, 0)

        def zcp(i, _):
            pltpu.sync_copy(zbuf, acc.at[pl.ds(s * r_t + i * ZR, ZR)])
            return 0
        lax.fori_loop(0, r_t // ZR, zcp, 0)
        plsc.subcore_barrier()

        ebase = s * e_t

        def load_idx(ch, ph):
            base = ebase + ch * ECH
            return [pltpu.async_copy(srcf_h.at[pl.ds(base, ECH)],
                                     srcb.at[ph], sem3),
                    pltpu.async_copy(ewf_h.at[pl.ds(base, ECH)],
                                     ewb.at[ph], sem3),
                    pltpu.async_copy(dst2d_h.at[pl.ds(base // 128, n_sub)],
                                     didx.at[pl.ds(ph * n_sub, n_sub)], sem3)]

        def wait_idx(ch, ph):
            base = ebase + ch * ECH
            pltpu.make_async_copy(srcf_h.at[pl.ds(base, ECH)],
                                  srcb.at[ph], sem3).wait()
            pltpu.make_async_copy(ewf_h.at[pl.ds(base, ECH)],
                                  ewb.at[ph], sem3).wait()
            pltpu.make_async_copy(dst2d_h.at[pl.ds(base // 128, n_sub)],
                                  didx.at[pl.ds(ph * n_sub, n_sub)],
                                  sem3).wait()

        def fire_gather(ph):
            def fill(i, _):
                sv = srcb[ph, pl.ds(i * L, L)]
                j = i // 8
                lo = (i % 8) * L
                gidx[ph * n_sub + j, pl.ds(lo, L)] = sv + coff
                return 0
            lax.fori_loop(0, ECH // L, fill, 0)
            for j in range(n_sub):
                pltpu.async_copy(
                    g_h.at[gidx.at[ph * n_sub + j]],
                    rowsb.at[pl.ds((ph * n_sub + j) * 128, 128)], sem)

        def wait_gather(ph):
            for j in range(n_sub):
                pltpu.make_async_copy(
                    g_h.at[gidx.at[ph * n_sub + j]],
                    rowsb.at[pl.ds((ph * n_sub + j) * 128, 128)], sem).wait()

        def scale(ph):
            def grp(i, _):
                wv = ewb[ph, pl.ds(i * L, L)]
                for l in range(L):
                    nb = _vbcast(wv, l)
                    r = ph * ECH + i * L + l
                    rowsb[r, pl.ds(0, L)] = rowsb[r, pl.ds(0, L)] * nb
                    rowsb[r, pl.ds(L, L)] = rowsb[r, pl.ds(L, L)] * nb
                return 0
            lax.fori_loop(0, ECH // L, grp, 0)

        def fire_scatter(ph):
            for j in range(n_sub):
                pltpu.async_copy(
                    rowsb.at[pl.ds((ph * n_sub + j) * 128, 128)],
                    acc.at[didx.at[ph * n_sub + j]], sem2, add=True)

        def wait_scatter(ph):
            for j in range(n_sub):
                pltpu.make_async_copy(
                    rowsb.at[pl.ds((ph * n_sub + j) * 128, 128)],
                    acc.at[didx.at[ph * n_sub + j]], sem2).wait()

        for d in load_idx(0, 0):
            d.wait()
        fire_gather(0)
        load_idx(1, 1)
        wait_gather(0)
        scale(0)
        fire_scatter(0)
        wait_idx(1, 1)
        fire_gather(1)

        def body(ch, _):
            ph = lax.rem(ch, 2)
            nxt = 1 - ph
            wait_scatter(nxt)
            load_idx(ch + 1, nxt)
            wait_gather(ph)
            scale(ph)
            fire_scatter(ph)
            wait_idx(ch + 1, nxt)
            fire_gather(nxt)
            return 0
        lax.fori_loop(1, n_ch - 1, body, 0)

        phl = (n_ch - 1) % 2
        wait_scatter(1 - phl)
        wait_gather(phl)
        scale(phl)
        fire_scatter(phl)
        wait_scatter(phl)

        plsc.subcore_barrier()

        WCH = 224
        def wout(i, _):
            r0 = s * r_t + i * WCH
            pltpu.sync_copy(acc.at[pl.ds(r0, WCH)],
                            rowsb.at[pl.ds(0, WCH)])
            pltpu.sync_copy(hs_h.at[pl.ds(coff + r0, WCH)],
                            rowsb.at[pl.ds(256, WCH)])
            pltpu.sync_copy(dinv_h.at[pl.ds(r0, WCH)],
                            ewb.at[0, pl.ds(0, WCH)])

            def wgrp(i2, _):
                dv = ewb[0, pl.ds(i2 * L, L)]
                for l in range(L):
                    db = _vbcast(dv, l)
                    r = i2 * L + l
                    rowsb[r, pl.ds(0, L)] = (
                        rowsb[r, pl.ds(0, L)] * db
                        + rowsb[256 + r, pl.ds(0, L)])
                    rowsb[r, pl.ds(L, L)] = (
                        rowsb[r, pl.ds(L, L)] * db
                        + rowsb[256 + r, pl.ds(L, L)])
                return 0
            lax.fori_loop(0, WCH // L, wgrp, 0)

            pltpu.sync_copy(rowsb.at[pl.ds(0, WCH)],
                            out_h.at[pl.ds(coff + r0, WCH)])
            return 0
        lax.fori_loop(0, r_t // WCH, wout, 0)

    return k


def _make_mlp(Np, BR=1024):
    grid = Np // BR

    def body(p_ref, dinv_ref, d2_ref, w1_ref, b1_ref, w2_ref,
             b2_ref, g2_ref, hs2_ref):
        dv = dinv_ref[...]
        d2 = d2_ref[...]
        x = jnp.concatenate([p_ref[0], p_ref[1]], axis=1)
        h = jnp.dot(x, w1_ref[...], precision=lax.Precision.HIGHEST,
                    preferred_element_type=jnp.float32) + b1_ref[...]
        h = jnp.maximum(h, 0.0)
        h2 = jnp.dot(h, w2_ref[...], precision=lax.Precision.HIGHEST,
                     preferred_element_type=jnp.float32)
        g2 = h2 * dv
        hs2 = h2 * d2 + b2_ref[...]
        g2_ref[0] = g2[:, :FH]
        g2_ref[1] = g2[:, FH:]
        hs2_ref[0] = hs2[:, :FH]
        hs2_ref[1] = hs2[:, FH:]

    return pl.pallas_call(
        body,
        grid=(grid,),
        in_specs=[pl.BlockSpec((2, BR, FH), lambda i: (0, i, 0)),
                  pl.BlockSpec((BR, 1), lambda i: (i, 0)),
                  pl.BlockSpec((BR, 1), lambda i: (i, 0)),
                  pl.BlockSpec((F, 2 * F), lambda i: (0, 0)),
                  pl.BlockSpec((1, 2 * F), lambda i: (0, 0)),
                  pl.BlockSpec((2 * F, F), lambda i: (0, 0)),
                  pl.BlockSpec((1, F), lambda i: (0, 0))],
        out_specs=[pl.BlockSpec((2, BR, FH), lambda i: (0, i, 0)),
                   pl.BlockSpec((2, BR, FH), lambda i: (0, i, 0))],
        out_shape=[jax.ShapeDtypeStruct((2, Np, FH), jnp.float32),
                   jax.ShapeDtypeStruct((2, Np, FH), jnp.float32)],
    )


def _make_segmax(Np, G):
    r_t = Np // NS
    TG = G + 8

    @functools.partial(
        pl.kernel,
        out_type=jax.ShapeDtypeStruct((2 * G, FH), jnp.float32),
        mesh=_mesh(),
        compiler_params=_sc_params(),
        scratch_types=[
            pltpu.VMEM((r_t, FH), jnp.float32),
            pltpu.VMEM((r_t,), jnp.int32),
            pltpu.VMEM((TG, FH), jnp.float32),
            pltpu.VMEM((8, FH), jnp.float32),
            pltpu.VMEM((8, FH), jnp.float32),
            pltpu.VMEM_SHARED((NS, G, FH), jnp.float32),
            pltpu.SemaphoreType.DMA,
        ],
    )
    def k(p_h, batch_h, out_h, rowsb, batchb, tb, racc, rtmp, shared, sem):
        c = lax.axis_index("c")
        s = lax.axis_index("s")
        coff = c * Np

        pltpu.sync_copy(p_h.at[pl.ds(coff + s * r_t, r_t)], rowsb)
        pltpu.sync_copy(batch_h.at[pl.ds(s * r_t, r_t)], batchb)

        ninf = jnp.full((L,), -jnp.inf, jnp.float32)

        def zt(i, _):
            tb[i, pl.ds(0, L)] = ninf
            tb[i, pl.ds(L, L)] = ninf
            return 0
        lax.fori_loop(0, TG, zt, 0)

        iota = lax.iota(jnp.int32, L)

        def grp(i, _):
            bv = batchb[pl.ds(i * L, L)]
            for l in range(L):
                gb = _vbcast(bv, l)
                r = i * L + l
                r0 = rowsb[r, pl.ds(0, L)]
                r1 = rowsb[r, pl.ds(L, L)]
                cur0 = plsc.load_gather(tb, [gb, iota])
                cur1 = plsc.load_gather(tb, [gb, iota + L])
                plsc.store_scatter(tb, [gb, iota], jnp.maximum(cur0, r0))
                plsc.store_scatter(tb, [gb, iota + L], jnp.maximum(cur1, r1))
            return 0
        lax.fori_loop(0, r_t // L, grp, 0)

        pltpu.sync_copy(tb.at[pl.ds(0, G)], shared.at[s])
        plsc.subcore_barrier()

        gs = G // NS
        pltpu.sync_copy(shared.at[0, pl.ds(s * gs, gs)], racc)

        for t in range(1, NS):
            pltpu.sync_copy(shared.at[t, pl.ds(s * gs, gs)], rtmp)
            for r in range(gs):
                for j in (0, L):
                    racc[r, pl.ds(j, L)] = jnp.maximum(
                        racc[r, pl.ds(j, L)], rtmp[r, pl.ds(j, L)])

        pltpu.sync_copy(racc, out_h.at[pl.ds(c * G + s * gs, gs)])

    return k


def kernel(x, edge_index, edge_attr, batch, embed, W1, b1, W2, b2):
    N = x.shape[0]
    E = edge_index.shape[1]
    V = embed.shape[0]
    G = 128

    Np = 50176
    Ep = 802816
    assert N <= Np and E <= Ep

    pad_n = Np - N
    pad_e = Ep - E

    xp = jnp.concatenate([x, (jnp.arange(pad_n, dtype=jnp.int32) % V)])
    srcf = jnp.concatenate(
        [edge_index[0], (jnp.arange(pad_e, dtype=jnp.int32) * 131) % N])
    dstf = jnp.concatenate(
        [edge_index[1], (jnp.arange(pad_e, dtype=jnp.int32) * 137) % N])
    ewf = jnp.concatenate([edge_attr, jnp.zeros((pad_e,), jnp.float32)])
    batchp = jnp.concatenate([batch, jnp.full((pad_n,), G, jnp.int32)])
    dst2d = dstf.reshape(Ep // 128, 128)

    embed2 = embed.reshape(V * 2, FH)

    h0_st, degp = _make_gather_deg(Np, Ep)(embed2, xp, dstf, ewf)

    dinv2d, d2_2d = _make_dinv(Np)(degp.reshape(NW, Np // 128, 128))
    dinvcol = dinv2d.reshape(Np, 1)
    d2col = d2_2d.reshape(Np, 1)

    g0_3, hs0_3 = _make_prep(Np)(h0_st.reshape(2, Np, FH), dinvcol, d2col)

    conv = _make_conv(Np, Ep)
    dinvf = dinv2d.reshape(Np)

    p0_st = conv(g0_3.reshape(2 * Np, FH), srcf, dst2d, ewf,
                 dinvf, hs0_3.reshape(2 * Np, FH))

    g2_3, hs2_3 = _make_mlp(Np)(p0_st.reshape(2, Np, FH),
                                dinvcol, d2col,
                                W1, b1.reshape(1, 2 * F), W2,
                                b2.reshape(1, F))

    p2_st = conv(g2_3.reshape(2 * Np, FH), srcf, dst2d, ewf,
                 dinvf, hs2_3.reshape(2 * Np, FH))

    out_st = _make_segmax(Np, G)(p2_st, batchp)

    out3 = out_st.reshape(2, G, FH)
    return jnp.concatenate([out3[0], out3[1]], axis=1)


# speedup vs baseline: 23.7112x; 1.1983x over previous
import functools

import jax
import jax.numpy as jnp
from jax import lax
from jax.experimental import pallas as pl
from jax.experimental.pallas import tpu as pltpu
from jax.experimental.pallas import tpu_sc as plsc

NC = 2
NS = 16
NW = NC * NS
L = 16

F = 64
FH = F // 2


def _mesh():
    return plsc.VectorSubcoreMesh(core_axis_name="c", subcore_axis_name="s")


def _sc_params():
    return pltpu.CompilerParams(
        needs_layout_passes=False, use_tc_tiling_on_sc=False)


def _vbcast(v, l):
    idx = jnp.full((L,), l, dtype=jnp.int32)
    dnums = lax.GatherDimensionNumbers(
        offset_dims=(), collapsed_slice_dims=(0,), start_index_map=(0,))
    return lax.gather(v, idx[:, None], dnums, (1,),
                      mode=lax.GatherScatterMode.PROMISE_IN_BOUNDS)


def _make_gather_deg(Np, Ep):
    n_w = Np // NW
    GCH = 112
    n_ch = n_w // GCH
    e_w = Ep // NW
    ECH = 1792
    e_ch = e_w // ECH

    @functools.partial(
        pl.kernel,
        out_type=(
            jax.ShapeDtypeStruct((2 * Np, FH), jnp.float32),
            jax.ShapeDtypeStruct((NW, Np), jnp.float32),
        ),
        mesh=_mesh(),
        compiler_params=_sc_params(),
        scratch_types=[
            pltpu.VMEM((n_w,), jnp.int32),
            pltpu.VMEM((GCH,), jnp.int32),
            pltpu.VMEM((GCH, FH), jnp.float32),
            pltpu.VMEM((Np,), jnp.float32),
            pltpu.VMEM((ECH,), jnp.int32),
            pltpu.VMEM((ECH,), jnp.float32),
            pltpu.SemaphoreType.DMA,
        ],
    )
    def k(embed2_h, xp_h, dstf_h, ewf_h, h0_h, degp_h,
          xb, idxb, rowsb, degb, dstb, ewb, sem):
        c = lax.axis_index("c")
        s = lax.axis_index("s")
        wid = s * NC + c
        nbase = wid * n_w

        pltpu.sync_copy(xp_h.at[pl.ds(nbase, n_w)], xb)

        def gather_half(off, out_base):
            def chunk(ch, _):
                def fill(i, _):
                    v = xb[pl.ds(ch * GCH + i * L, L)]
                    idxb[pl.ds(i * L, L)] = v * 2 + off
                    return 0
                lax.fori_loop(0, GCH // L, fill, 0)
                pltpu.async_copy(embed2_h.at[idxb], rowsb, sem).wait()
                pltpu.sync_copy(
                    rowsb, h0_h.at[pl.ds(out_base + ch * GCH, GCH)])
                return 0
            lax.fori_loop(0, n_ch, chunk, 0)

        gather_half(0, nbase)
        gather_half(1, Np + nbase)

        def zero(i, _):
            degb[pl.ds(i * L, L)] = jnp.zeros((L,), jnp.float32)
            return 0
        lax.fori_loop(0, Np // L, zero, 0)

        ebase = wid * e_w

        def echunk(ch, _):
            pltpu.sync_copy(dstf_h.at[pl.ds(ebase + ch * ECH, ECH)], dstb)
            pltpu.sync_copy(ewf_h.at[pl.ds(ebase + ch * ECH, ECH)], ewb)

            def grp(i, _):
                di = dstb[pl.ds(i * L, L)]
                wv = ewb[pl.ds(i * L, L)]
                plsc.addupdate_scatter(degb, [di], wv)
                return 0
            lax.fori_loop(0, ECH // L, grp, 0)
            return 0
        lax.fori_loop(0, e_ch, echunk, 0)

        pltpu.sync_copy(degb, degp_h.at[wid])

    return k


def _make_dinv(Np):
    R = Np // 128
    BR = 8
    grid = R // BR

    def body(degp_ref, dinv_ref):
        deg = jnp.sum(degp_ref[...], axis=0) + 1.0
        dinv_ref[...] = jnp.where(deg > 0, lax.rsqrt(deg), 0.0)

    return pl.pallas_call(
        body,
        grid=(grid,),
        in_specs=[pl.BlockSpec((NW, BR, 128), lambda i: (0, i, 0))],
        out_specs=[pl.BlockSpec((BR, 128), lambda i: (i, 0))],
        out_shape=[jax.ShapeDtypeStruct((R, 128), jnp.float32)],
    )


def _make_ews(Np, Ep):
    e_w = Ep // NW
    ECH = 1792
    n_ch = e_w // ECH

    @functools.partial(
        pl.kernel,
        out_type=jax.ShapeDtypeStruct((Ep,), jnp.float32),
        mesh=_mesh(),
        compiler_params=_sc_params(),
        scratch_types=[
            pltpu.VMEM((Np,), jnp.float32),
            pltpu.VMEM((ECH,), jnp.int32),
            pltpu.VMEM((ECH,), jnp.float32),
            pltpu.VMEM((ECH,), jnp.float32),
        ],
    )
    def k(dinv_h, srcf_h, ewf_h, ews_h, dv, srcb, ewb, ewsb):
        c = lax.axis_index("c")
        s = lax.axis_index("s")
        wid = s * NC + c
        ebase = wid * e_w
        pltpu.sync_copy(dinv_h, dv)

        def chunk(ch, _):
            base = ebase + ch * ECH
            pltpu.sync_copy(srcf_h.at[pl.ds(base, ECH)], srcb)
            pltpu.sync_copy(ewf_h.at[pl.ds(base, ECH)], ewb)

            def grp(i, _):
                sv = srcb[pl.ds(i * L, L)]
                ewsb[pl.ds(i * L, L)] = (ewb[pl.ds(i * L, L)]
                                         * plsc.load_gather(dv, [sv]))
                return 0
            lax.fori_loop(0, ECH // L, grp, 0)
            pltpu.sync_copy(ewsb, ews_h.at[pl.ds(base, ECH)])
            return 0
        lax.fori_loop(0, n_ch, chunk, 0)

    return k


def _make_conv(Np, Ep):
    e_t = Ep // NS
    ECH = 256
    n_sub = ECH // 128
    n_ch = e_t // ECH
    r_t = Np // NS
    ZR = 196

    @functools.partial(
        pl.kernel,
        out_type=jax.ShapeDtypeStruct((2 * Np, FH), jnp.float32),
        mesh=_mesh(),
        compiler_params=_sc_params(),
        scratch_types=[
            pltpu.VMEM((2, ECH), jnp.int32),
            pltpu.VMEM((2, ECH), jnp.float32),
            pltpu.VMEM((2 * n_sub, 128), jnp.int32),
            pltpu.VMEM((2 * n_sub, 128), jnp.int32),
            pltpu.VMEM((2 * ECH, FH), jnp.float32),
            pltpu.VMEM((ZR, FH), jnp.float32),
            pltpu.VMEM((F,), jnp.float32),
            pltpu.VMEM_SHARED((Np, FH), jnp.float32),
            pltpu.SemaphoreType.DMA,
            pltpu.SemaphoreType.DMA,
            pltpu.SemaphoreType.DMA,
        ],
    )
    def k(g_h, srcf_h, dst2d_h, ewf_h, dinv_h, bias_h, out_h,
          srcb, ewb, gidx, didx, rowsb, zbuf, biasb, acc, sem, sem2, sem3):
        c = lax.axis_index("c")
        s = lax.axis_index("s")
        coff = c * Np

        def zf(i, _):
            zbuf[i, pl.ds(0, L)] = jnp.zeros((L,), jnp.float32)
            zbuf[i, pl.ds(L, L)] = jnp.zeros((L,), jnp.float32)
            return 0
        lax.fori_loop(0, ZR, zf, 0)

        def zcp(i, _):
            pltpu.sync_copy(zbuf, acc.at[pl.ds(s * r_t + i * ZR, ZR)])
            return 0
        lax.fori_loop(0, r_t // ZR, zcp, 0)
        plsc.subcore_barrier()

        ebase = s * e_t

        def load_idx(ch, ph):
            base = ebase + ch * ECH
            return [pltpu.async_copy(srcf_h.at[pl.ds(base, ECH)],
                                     srcb.at[ph], sem3),
                    pltpu.async_copy(ewf_h.at[pl.ds(base, ECH)],
                                     ewb.at[ph], sem3),
                    pltpu.async_copy(dst2d_h.at[pl.ds(base // 128, n_sub)],
                                     didx.at[pl.ds(ph * n_sub, n_sub)], sem3)]

        def wait_idx(ch, ph):
            base = ebase + ch * ECH
            pltpu.make_async_copy(srcf_h.at[pl.ds(base, ECH)],
                                  srcb.at[ph], sem3).wait()
            pltpu.make_async_copy(ewf_h.at[pl.ds(base, ECH)],
                                  ewb.at[ph], sem3).wait()
            pltpu.make_async_copy(dst2d_h.at[pl.ds(base // 128, n_sub)],
                                  didx.at[pl.ds(ph * n_sub, n_sub)],
                                  sem3).wait()

        def fire_gather(ph):
            def fill(i, _):
                sv = srcb[ph, pl.ds(i * L, L)]
                j = i // 8
                lo = (i % 8) * L
                gidx[ph * n_sub + j, pl.ds(lo, L)] = sv + coff
                return 0
            lax.fori_loop(0, ECH // L, fill, 0)
            for j in range(n_sub):
                pltpu.async_copy(
                    g_h.at[gidx.at[ph * n_sub + j]],
                    rowsb.at[pl.ds((ph * n_sub + j) * 128, 128)], sem)

        def wait_gather(ph):
            for j in range(n_sub):
                pltpu.make_async_copy(
                    g_h.at[gidx.at[ph * n_sub + j]],
                    rowsb.at[pl.ds((ph * n_sub + j) * 128, 128)], sem).wait()

        def scale(ph):
            def grp(i, _):
                wv = ewb[ph, pl.ds(i * L, L)]
                for l in range(L):
                    nb = _vbcast(wv, l)
                    r = ph * ECH + i * L + l
                    rowsb[r, pl.ds(0, L)] = rowsb[r, pl.ds(0, L)] * nb
                    rowsb[r, pl.ds(L, L)] = rowsb[r, pl.ds(L, L)] * nb
                return 0
            lax.fori_loop(0, ECH // L, grp, 0)

        def fire_scatter(ph):
            for j in range(n_sub):
                pltpu.async_copy(
                    rowsb.at[pl.ds((ph * n_sub + j) * 128, 128)],
                    acc.at[didx.at[ph * n_sub + j]], sem2, add=True)

        def wait_scatter(ph):
            for j in range(n_sub):
                pltpu.make_async_copy(
                    rowsb.at[pl.ds((ph * n_sub + j) * 128, 128)],
                    acc.at[didx.at[ph * n_sub + j]], sem2).wait()

        for d in load_idx(0, 0):
            d.wait()
        fire_gather(0)
        load_idx(1, 1)
        wait_gather(0)
        scale(0)
        fire_scatter(0)
        wait_idx(1, 1)
        fire_gather(1)

        def body(ch, _):
            ph = lax.rem(ch, 2)
            nxt = 1 - ph
            wait_scatter(nxt)
            load_idx(ch + 1, nxt)
            wait_gather(ph)
            scale(ph)
            fire_scatter(ph)
            wait_idx(ch + 1, nxt)
            fire_gather(nxt)
            return 0
        lax.fori_loop(1, n_ch - 1, body, 0)

        phl = (n_ch - 1) % 2
        wait_scatter(1 - phl)
        wait_gather(phl)
        scale(phl)
        fire_scatter(phl)
        wait_scatter(phl)

        plsc.subcore_barrier()

        pltpu.sync_copy(bias_h, biasb)
        b_lo = c * FH
        WCH = 224
        def wout(i, _):
            r0 = s * r_t + i * WCH
            pltpu.sync_copy(acc.at[pl.ds(r0, WCH)],
                            rowsb.at[pl.ds(0, WCH)])
            pltpu.sync_copy(g_h.at[pl.ds(coff + r0, WCH)],
                            rowsb.at[pl.ds(256, WCH)])
            pltpu.sync_copy(dinv_h.at[pl.ds(r0, WCH)],
                            ewb.at[0, pl.ds(0, WCH)])

            bv0 = biasb[pl.ds(b_lo, L)]
            bv1 = biasb[pl.ds(b_lo + L, L)]

            def wgrp(i2, _):
                dv = ewb[0, pl.ds(i2 * L, L)]
                for l in range(L):
                    db = _vbcast(dv, l)
                    d2b = db * db
                    r = i2 * L + l
                    rowsb[r, pl.ds(0, L)] = (
                        rowsb[r, pl.ds(0, L)] * db
                        + rowsb[256 + r, pl.ds(0, L)] * d2b + bv0)
                    rowsb[r, pl.ds(L, L)] = (
                        rowsb[r, pl.ds(L, L)] * db
                        + rowsb[256 + r, pl.ds(L, L)] * d2b + bv1)
                return 0
            lax.fori_loop(0, WCH // L, wgrp, 0)

            pltpu.sync_copy(rowsb.at[pl.ds(0, WCH)],
                            out_h.at[pl.ds(coff + r0, WCH)])
            return 0
        lax.fori_loop(0, r_t // WCH, wout, 0)

    return k


def _make_mlp(Np, BR=1024):
    grid = Np // BR

    def body(p_ref, w1_ref, b1_ref, w2_ref, h2_ref):
        x = jnp.concatenate([p_ref[0], p_ref[1]], axis=1)
        h = jnp.dot(x, w1_ref[...], precision=lax.Precision.HIGHEST,
                    preferred_element_type=jnp.float32) + b1_ref[...]
        h = jnp.maximum(h, 0.0)
        h2 = jnp.dot(h, w2_ref[...], precision=lax.Precision.HIGHEST,
                     preferred_element_type=jnp.float32)
        h2_ref[0] = h2[:, :FH]
        h2_ref[1] = h2[:, FH:]

    return pl.pallas_call(
        body,
        grid=(grid,),
        in_specs=[pl.BlockSpec((2, BR, FH), lambda i: (0, i, 0)),
                  pl.BlockSpec((F, 2 * F), lambda i: (0, 0)),
                  pl.BlockSpec((1, 2 * F), lambda i: (0, 0)),
                  pl.BlockSpec((2 * F, F), lambda i: (0, 0))],
        out_specs=[pl.BlockSpec((2, BR, FH), lambda i: (0, i, 0))],
        out_shape=[jax.ShapeDtypeStruct((2, Np, FH), jnp.float32)],
    )


def _make_segmax(Np, G):
    r_t = Np // NS
    TG = G + 8

    @functools.partial(
        pl.kernel,
        out_type=jax.ShapeDtypeStruct((2 * G, FH), jnp.float32),
        mesh=_mesh(),
        compiler_params=_sc_params(),
        scratch_types=[
            pltpu.VMEM((r_t, FH), jnp.float32),
            pltpu.VMEM((r_t,), jnp.int32),
            pltpu.VMEM((TG, FH), jnp.float32),
            pltpu.VMEM((8, FH), jnp.float32),
            pltpu.VMEM((8, FH), jnp.float32),
            pltpu.VMEM_SHARED((NS, G, FH), jnp.float32),
            pltpu.SemaphoreType.DMA,
        ],
    )
    def k(p_h, batch_h, out_h, rowsb, batchb, tb, racc, rtmp, shared, sem):
        c = lax.axis_index("c")
        s = lax.axis_index("s")
        coff = c * Np

        pltpu.sync_copy(p_h.at[pl.ds(coff + s * r_t, r_t)], rowsb)
        pltpu.sync_copy(batch_h.at[pl.ds(s * r_t, r_t)], batchb)

        ninf = jnp.full((L,), -jnp.inf, jnp.float32)

        def zt(i, _):
            tb[i, pl.ds(0, L)] = ninf
            tb[i, pl.ds(L, L)] = ninf
            return 0
        lax.fori_loop(0, TG, zt, 0)

        iota = lax.iota(jnp.int32, L)

        def grp(i, _):
            bv = batchb[pl.ds(i * L, L)]
            for l in range(L):
                gb = _vbcast(bv, l)
                r = i * L + l
                r0 = rowsb[r, pl.ds(0, L)]
                r1 = rowsb[r, pl.ds(L, L)]
                cur0 = plsc.load_gather(tb, [gb, iota])
                cur1 = plsc.load_gather(tb, [gb, iota + L])
                plsc.store_scatter(tb, [gb, iota], jnp.maximum(cur0, r0))
                plsc.store_scatter(tb, [gb, iota + L], jnp.maximum(cur1, r1))
            return 0
        lax.fori_loop(0, r_t // L, grp, 0)

        pltpu.sync_copy(tb.at[pl.ds(0, G)], shared.at[s])
        plsc.subcore_barrier()

        gs = G // NS
        pltpu.sync_copy(shared.at[0, pl.ds(s * gs, gs)], racc)

        for t in range(1, NS):
            pltpu.sync_copy(shared.at[t, pl.ds(s * gs, gs)], rtmp)
            for r in range(gs):
                for j in (0, L):
                    racc[r, pl.ds(j, L)] = jnp.maximum(
                        racc[r, pl.ds(j, L)], rtmp[r, pl.ds(j, L)])

        pltpu.sync_copy(racc, out_h.at[pl.ds(c * G + s * gs, gs)])

    return k


def kernel(x, edge_index, edge_attr, batch, embed, W1, b1, W2, b2):
    N = x.shape[0]
    E = edge_index.shape[1]
    V = embed.shape[0]
    G = 128

    Np = 50176
    Ep = 802816
    assert N <= Np and E <= Ep

    pad_n = Np - N
    pad_e = Ep - E

    xp = jnp.concatenate([x, (jnp.arange(pad_n, dtype=jnp.int32) % V)])
    srcf = jnp.concatenate(
        [edge_index[0], (jnp.arange(pad_e, dtype=jnp.int32) * 131) % N])
    dstf = jnp.concatenate(
        [edge_index[1], (jnp.arange(pad_e, dtype=jnp.int32) * 137) % N])
    ewf = jnp.concatenate([edge_attr, jnp.zeros((pad_e,), jnp.float32)])
    batchp = jnp.concatenate([batch, jnp.full((pad_n,), G, jnp.int32)])
    dst2d = dstf.reshape(Ep // 128, 128)

    embed2 = embed.reshape(V * 2, FH)

    h0_st, degp = _make_gather_deg(Np, Ep)(embed2, xp, dstf, ewf)

    (dinv2d,) = _make_dinv(Np)(degp.reshape(NW, Np // 128, 128))
    dinvf = dinv2d.reshape(Np)

    ews = _make_ews(Np, Ep)(dinvf, srcf, ewf)

    conv = _make_conv(Np, Ep)
    zbias = jnp.zeros((F,), jnp.float32)

    p0_st = conv(h0_st, srcf, dst2d, ews, dinvf, zbias)

    (h2_3,) = _make_mlp(Np)(p0_st.reshape(2, Np, FH),
                            W1, b1.reshape(1, 2 * F), W2)

    p2_st = conv(h2_3.reshape(2 * Np, FH), srcf, dst2d, ews, dinvf, b2)

    out_st = _make_segmax(Np, G)(p2_st, batchp)

    out3 = out_st.reshape(2, G, FH)
    return jnp.concatenate([out3[0], out3[1]], axis=1)


# speedup vs baseline: 29.1273x; 1.2284x over previous
import functools

import jax
import jax.numpy as jnp
from jax import lax
from jax.experimental import pallas as pl
from jax.experimental.pallas import tpu as pltpu
from jax.experimental.pallas import tpu_sc as plsc

NC = 2
NS = 16
NW = NC * NS
L = 16

F = 64
FH = F // 2


def _mesh():
    return plsc.VectorSubcoreMesh(core_axis_name="c", subcore_axis_name="s")


def _sc_params():
    return pltpu.CompilerParams(
        needs_layout_passes=False, use_tc_tiling_on_sc=False)


def _vbcast(v, l):
    idx = jnp.full((L,), l, dtype=jnp.int32)
    dnums = lax.GatherDimensionNumbers(
        offset_dims=(), collapsed_slice_dims=(0,), start_index_map=(0,))
    return lax.gather(v, idx[:, None], dnums, (1,),
                      mode=lax.GatherScatterMode.PROMISE_IN_BOUNDS)


def _make_gather_deg(Np, Ep):
    n_w = Np // NW
    GCH = 112
    n_ch = n_w // GCH
    e_w = Ep // NW
    ECH = 1792
    e_ch = e_w // ECH

    @functools.partial(
        pl.kernel,
        out_type=(
            jax.ShapeDtypeStruct((2 * Np, FH), jnp.float32),
            jax.ShapeDtypeStruct((NW, Np), jnp.float32),
        ),
        mesh=_mesh(),
        compiler_params=_sc_params(),
        scratch_types=[
            pltpu.VMEM((n_w,), jnp.int32),
            pltpu.VMEM((GCH,), jnp.int32),
            pltpu.VMEM((GCH, FH), jnp.float32),
            pltpu.VMEM((Np,), jnp.float32),
            pltpu.VMEM((ECH,), jnp.int32),
            pltpu.VMEM((ECH,), jnp.float32),
            pltpu.SemaphoreType.DMA,
        ],
    )
    def k(embed2_h, xp_h, dstf_h, ewf_h, h0_h, degp_h,
          xb, idxb, rowsb, degb, dstb, ewb, sem):
        c = lax.axis_index("c")
        s = lax.axis_index("s")
        wid = s * NC + c
        nbase = wid * n_w

        pltpu.sync_copy(xp_h.at[pl.ds(nbase, n_w)], xb)

        def gather_half(off, out_base):
            def chunk(ch, _):
                def fill(i, _):
                    v = xb[pl.ds(ch * GCH + i * L, L)]
                    idxb[pl.ds(i * L, L)] = v * 2 + off
                    return 0
                lax.fori_loop(0, GCH // L, fill, 0)
                pltpu.async_copy(embed2_h.at[idxb], rowsb, sem).wait()
                pltpu.sync_copy(
                    rowsb, h0_h.at[pl.ds(out_base + ch * GCH, GCH)])
                return 0
            lax.fori_loop(0, n_ch, chunk, 0)

        gather_half(0, nbase)
        gather_half(1, Np + nbase)

        def zero(i, _):
            degb[pl.ds(i * L, L)] = jnp.zeros((L,), jnp.float32)
            return 0
        lax.fori_loop(0, Np // L, zero, 0)

        ebase = wid * e_w

        def echunk(ch, _):
            pltpu.sync_copy(dstf_h.at[pl.ds(ebase + ch * ECH, ECH)], dstb)
            pltpu.sync_copy(ewf_h.at[pl.ds(ebase + ch * ECH, ECH)], ewb)

            def grp(i, _):
                di = dstb[pl.ds(i * L, L)]
                wv = ewb[pl.ds(i * L, L)]
                plsc.addupdate_scatter(degb, [di], wv)
                return 0
            lax.fori_loop(0, ECH // L, grp, 0)
            return 0
        lax.fori_loop(0, e_ch, echunk, 0)

        pltpu.sync_copy(degb, degp_h.at[wid])

    return k


def _make_dinv(Np):
    R = Np // 128
    BR = 8
    grid = R // BR

    def body(degp_ref, dinv_ref):
        deg = jnp.sum(degp_ref[...], axis=0) + 1.0
        dinv_ref[...] = jnp.where(deg > 0, lax.rsqrt(deg), 0.0)

    return pl.pallas_call(
        body,
        grid=(grid,),
        in_specs=[pl.BlockSpec((NW, BR, 128), lambda i: (0, i, 0))],
        out_specs=[pl.BlockSpec((BR, 128), lambda i: (i, 0))],
        out_shape=[jax.ShapeDtypeStruct((R, 128), jnp.float32)],
    )


def _make_ews(Np, Ep):
    e_w = Ep // NW
    ECH = 1792
    n_ch = e_w // ECH

    @functools.partial(
        pl.kernel,
        out_type=jax.ShapeDtypeStruct((Ep,), jnp.float32),
        mesh=_mesh(),
        compiler_params=_sc_params(),
        scratch_types=[
            pltpu.VMEM((Np,), jnp.float32),
            pltpu.VMEM((ECH,), jnp.int32),
            pltpu.VMEM((ECH,), jnp.float32),
            pltpu.VMEM((ECH,), jnp.float32),
        ],
    )
    def k(dinv_h, srcf_h, ewf_h, ews_h, dv, srcb, ewb, ewsb):
        c = lax.axis_index("c")
        s = lax.axis_index("s")
        wid = s * NC + c
        ebase = wid * e_w
        pltpu.sync_copy(dinv_h, dv)

        def chunk(ch, _):
            base = ebase + ch * ECH
            pltpu.sync_copy(srcf_h.at[pl.ds(base, ECH)], srcb)
            pltpu.sync_copy(ewf_h.at[pl.ds(base, ECH)], ewb)

            def grp(i, _):
                sv = srcb[pl.ds(i * L, L)]
                ewsb[pl.ds(i * L, L)] = (ewb[pl.ds(i * L, L)]
                                         * plsc.load_gather(dv, [sv]))
                return 0
            lax.fori_loop(0, ECH // L, grp, 0)
            pltpu.sync_copy(ewsb, ews_h.at[pl.ds(base, ECH)])
            return 0
        lax.fori_loop(0, n_ch, chunk, 0)

    return k


def _make_conv(Np, Ep):
    e_t = Ep // NS
    ECH = 256
    n_sub = ECH // 128
    n_ch = e_t // ECH
    assert n_ch % 3 == 1 and n_ch >= 7
    r_t = Np // NS

    @functools.partial(
        pl.kernel,
        out_type=jax.ShapeDtypeStruct((2 * Np, FH), jnp.float32),
        mesh=_mesh(),
        compiler_params=_sc_params(),
        scratch_types=[
            pltpu.VMEM((3, ECH), jnp.int32),
            pltpu.VMEM((3, ECH), jnp.float32),
            pltpu.VMEM((3 * n_sub, 128), jnp.int32),
            pltpu.VMEM((3 * n_sub, 128), jnp.int32),
            pltpu.VMEM((3 * ECH, FH), jnp.float32),
            pltpu.VMEM((F,), jnp.float32),
            pltpu.VMEM_SHARED((Np, FH), jnp.float32),
            pltpu.SemaphoreType.DMA,
            pltpu.SemaphoreType.DMA,
            pltpu.SemaphoreType.DMA,
            pltpu.SemaphoreType.DMA,
            pltpu.SemaphoreType.DMA,
            pltpu.SemaphoreType.DMA,
            pltpu.SemaphoreType.DMA,
        ],
    )
    def k(g_h, srcf_h, dst2d_h, ewf_h, dinv_h, bias_h, zrows_h, out_h,
          srcb, ewb, gidx, didx, rowsb, biasb, acc,
          sg0, sg1, sg2, ss0, ss1, ss2, sem3):
        c = lax.axis_index("c")
        s = lax.axis_index("s")
        coff = c * Np
        semg = (sg0, sg1, sg2)
        sems = (ss0, ss1, ss2)

        pltpu.sync_copy(zrows_h, acc.at[pl.ds(s * r_t, r_t)])
        plsc.subcore_barrier()

        ebase = s * e_t

        def load_idx(ch, ph):
            base = ebase + ch * ECH
            pltpu.async_copy(srcf_h.at[pl.ds(base, ECH)], srcb.at[ph], sem3)
            pltpu.async_copy(ewf_h.at[pl.ds(base, ECH)], ewb.at[ph], sem3)
            pltpu.async_copy(dst2d_h.at[pl.ds(base // 128, n_sub)],
                             didx.at[pl.ds(ph * n_sub, n_sub)], sem3)

        def wait_idx(ch, ph):
            base = ebase + ch * ECH
            pltpu.make_async_copy(srcf_h.at[pl.ds(base, ECH)],
                                  srcb.at[ph], sem3).wait()
            pltpu.make_async_copy(ewf_h.at[pl.ds(base, ECH)],
                                  ewb.at[ph], sem3).wait()
            pltpu.make_async_copy(dst2d_h.at[pl.ds(base // 128, n_sub)],
                                  didx.at[pl.ds(ph * n_sub, n_sub)],
                                  sem3).wait()

        def fire_gather(ph):
            def fill(i, _):
                sv = srcb[ph, pl.ds(i * L, L)]
                j = i // 8
                lo = (i % 8) * L
                gidx[ph * n_sub + j, pl.ds(lo, L)] = sv + coff
                return 0
            lax.fori_loop(0, ECH // L, fill, 0)
            for j in range(n_sub):
                pltpu.async_copy(
                    g_h.at[gidx.at[ph * n_sub + j]],
                    rowsb.at[pl.ds((ph * n_sub + j) * 128, 128)], semg[ph])

        def wait_gather(ph):
            for j in range(n_sub):
                pltpu.make_async_copy(
                    g_h.at[gidx.at[ph * n_sub + j]],
                    rowsb.at[pl.ds((ph * n_sub + j) * 128, 128)],
                    semg[ph]).wait()

        def scale(ph):
            def grp(i, _):
                wv = ewb[ph, pl.ds(i * L, L)]
                for l in range(L):
                    nb = _vbcast(wv, l)
                    r = ph * ECH + i * L + l
                    rowsb[r, pl.ds(0, L)] = rowsb[r, pl.ds(0, L)] * nb
                    rowsb[r, pl.ds(L, L)] = rowsb[r, pl.ds(L, L)] * nb
                return 0
            lax.fori_loop(0, ECH // L, grp, 0)

        def fire_scatter(ph):
            for j in range(n_sub):
                pltpu.async_copy(
                    rowsb.at[pl.ds((ph * n_sub + j) * 128, 128)],
                    acc.at[didx.at[ph * n_sub + j]], sems[ph], add=True)

        def wait_scatter(ph):
            for j in range(n_sub):
                pltpu.make_async_copy(
                    rowsb.at[pl.ds((ph * n_sub + j) * 128, 128)],
                    acc.at[didx.at[ph * n_sub + j]], sems[ph]).wait()

        def body_sub(ch, ph, phn):
            wait_scatter(phn)
            load_idx(ch + 2, phn)
            wait_gather(ph)
            scale(ph)
            fire_scatter(ph)
            wait_idx(ch + 2, phn)
            fire_gather(phn)

        load_idx(0, 0)
        wait_idx(0, 0)
        fire_gather(0)
        load_idx(1, 1)
        wait_idx(1, 1)
        fire_gather(1)
        load_idx(2, 2)
        wait_gather(0)
        scale(0)
        fire_scatter(0)
        wait_idx(2, 2)
        fire_gather(2)

        def pbody(p, _):
            ch = 3 * p
            body_sub(ch + 1, 1, 0)
            body_sub(ch + 2, 2, 1)
            body_sub(ch + 3, 0, 2)
            return 0
        lax.fori_loop(0, (n_ch - 4) // 3, pbody, 0)

        body_sub(n_ch - 3, 1, 0)
        wait_scatter(1)
        wait_gather(2)
        scale(2)
        fire_scatter(2)
        wait_scatter(2)
        wait_gather(0)
        scale(0)
        fire_scatter(0)
        wait_scatter(0)

        plsc.subcore_barrier()

        pltpu.sync_copy(bias_h, biasb)
        b_lo = c * FH
        WCH = 224
        def wout(i, _):
            r0 = s * r_t + i * WCH
            pltpu.sync_copy(acc.at[pl.ds(r0, WCH)],
                            rowsb.at[pl.ds(0, WCH)])
            pltpu.sync_copy(g_h.at[pl.ds(coff + r0, WCH)],
                            rowsb.at[pl.ds(256, WCH)])
            pltpu.sync_copy(dinv_h.at[pl.ds(r0, WCH)],
                            ewb.at[0, pl.ds(0, WCH)])

            bv0 = biasb[pl.ds(b_lo, L)]
            bv1 = biasb[pl.ds(b_lo + L, L)]

            def wgrp(i2, _):
                dv = ewb[0, pl.ds(i2 * L, L)]
                for l in range(L):
                    db = _vbcast(dv, l)
                    d2b = db * db
                    r = i2 * L + l
                    rowsb[r, pl.ds(0, L)] = (
                        rowsb[r, pl.ds(0, L)] * db
                        + rowsb[256 + r, pl.ds(0, L)] * d2b + bv0)
                    rowsb[r, pl.ds(L, L)] = (
                        rowsb[r, pl.ds(L, L)] * db
                        + rowsb[256 + r, pl.ds(L, L)] * d2b + bv1)
                return 0
            lax.fori_loop(0, WCH // L, wgrp, 0)

            pltpu.sync_copy(rowsb.at[pl.ds(0, WCH)],
                            out_h.at[pl.ds(coff + r0, WCH)])
            return 0
        lax.fori_loop(0, r_t // WCH, wout, 0)

    return k


def _make_mlp(Np, BR=1024):
    grid = Np // BR

    def body(p_ref, w1_ref, b1_ref, w2_ref, h2_ref):
        x = jnp.concatenate([p_ref[0], p_ref[1]], axis=1)
        h = jnp.dot(x, w1_ref[...], precision=lax.Precision.HIGHEST,
                    preferred_element_type=jnp.float32) + b1_ref[...]
        h = jnp.maximum(h, 0.0)
        h2 = jnp.dot(h, w2_ref[...], precision=lax.Precision.HIGHEST,
                     preferred_element_type=jnp.float32)
        h2_ref[0] = h2[:, :FH]
        h2_ref[1] = h2[:, FH:]

    return pl.pallas_call(
        body,
        grid=(grid,),
        in_specs=[pl.BlockSpec((2, BR, FH), lambda i: (0, i, 0)),
                  pl.BlockSpec((F, 2 * F), lambda i: (0, 0)),
                  pl.BlockSpec((1, 2 * F), lambda i: (0, 0)),
                  pl.BlockSpec((2 * F, F), lambda i: (0, 0))],
        out_specs=[pl.BlockSpec((2, BR, FH), lambda i: (0, i, 0))],
        out_shape=[jax.ShapeDtypeStruct((2, Np, FH), jnp.float32)],
    )


def _make_segmax(Np, G):
    r_t = Np // NS
    TG = G + 8

    @functools.partial(
        pl.kernel,
        out_type=jax.ShapeDtypeStruct((2 * G, FH), jnp.float32),
        mesh=_mesh(),
        compiler_params=_sc_params(),
        scratch_types=[
            pltpu.VMEM((r_t, FH), jnp.float32),
            pltpu.VMEM((r_t,), jnp.int32),
            pltpu.VMEM((TG, FH), jnp.float32),
            pltpu.VMEM((8, FH), jnp.float32),
            pltpu.VMEM((8, FH), jnp.float32),
            pltpu.VMEM_SHARED((NS, G, FH), jnp.float32),
            pltpu.SemaphoreType.DMA,
        ],
    )
    def k(p_h, batch_h, out_h, rowsb, batchb, tb, racc, rtmp, shared, sem):
        c = lax.axis_index("c")
        s = lax.axis_index("s")
        coff = c * Np

        pltpu.sync_copy(p_h.at[pl.ds(coff + s * r_t, r_t)], rowsb)
        pltpu.sync_copy(batch_h.at[pl.ds(s * r_t, r_t)], batchb)

        ninf = jnp.full((L,), -jnp.inf, jnp.float32)

        def zt(i, _):
            tb[i, pl.ds(0, L)] = ninf
            tb[i, pl.ds(L, L)] = ninf
            return 0
        lax.fori_loop(0, TG, zt, 0)

        iota = lax.iota(jnp.int32, L)

        def grp(i, _):
            bv = batchb[pl.ds(i * L, L)]
            for l in range(L):
                gb = _vbcast(bv, l)
                r = i * L + l
                r0 = rowsb[r, pl.ds(0, L)]
                r1 = rowsb[r, pl.ds(L, L)]
                cur0 = plsc.load_gather(tb, [gb, iota])
                cur1 = plsc.load_gather(tb, [gb, iota + L])
                plsc.store_scatter(tb, [gb, iota], jnp.maximum(cur0, r0))
                plsc.store_scatter(tb, [gb, iota + L], jnp.maximum(cur1, r1))
            return 0
        lax.fori_loop(0, r_t // L, grp, 0)

        pltpu.sync_copy(tb.at[pl.ds(0, G)], shared.at[s])
        plsc.subcore_barrier()

        gs = G // NS
        pltpu.sync_copy(shared.at[0, pl.ds(s * gs, gs)], racc)

        for t in range(1, NS):
            pltpu.sync_copy(shared.at[t, pl.ds(s * gs, gs)], rtmp)
            for r in range(gs):
                for j in (0, L):
                    racc[r, pl.ds(j, L)] = jnp.maximum(
                        racc[r, pl.ds(j, L)], rtmp[r, pl.ds(j, L)])

        pltpu.sync_copy(racc, out_h.at[pl.ds(c * G + s * gs, gs)])

    return k


def kernel(x, edge_index, edge_attr, batch, embed, W1, b1, W2, b2):
    N = x.shape[0]
    E = edge_index.shape[1]
    V = embed.shape[0]
    G = 128

    Np = 50176
    Ep = 802816
    assert N <= Np and E <= Ep

    pad_n = Np - N
    pad_e = Ep - E

    xp = jnp.concatenate([x, (jnp.arange(pad_n, dtype=jnp.int32) % V)])
    srcf = jnp.concatenate(
        [edge_index[0], (jnp.arange(pad_e, dtype=jnp.int32) * 131) % N])
    dstf = jnp.concatenate(
        [edge_index[1], (jnp.arange(pad_e, dtype=jnp.int32) * 137) % N])
    ewf = jnp.concatenate([edge_attr, jnp.zeros((pad_e,), jnp.float32)])
    batchp = jnp.concatenate([batch, jnp.full((pad_n,), G, jnp.int32)])
    dst2d = dstf.reshape(Ep // 128, 128)

    embed2 = embed.reshape(V * 2, FH)

    h0_st, degp = _make_gather_deg(Np, Ep)(embed2, xp, dstf, ewf)

    (dinv2d,) = _make_dinv(Np)(degp.reshape(NW, Np // 128, 128))
    dinvf = dinv2d.reshape(Np)

    ews = _make_ews(Np, Ep)(dinvf, srcf, ewf)

    conv = _make_conv(Np, Ep)
    zbias = jnp.zeros((F,), jnp.float32)
    zrows = jnp.zeros((Np // NS, FH), jnp.float32)

    p0_st = conv(h0_st, srcf, dst2d, ews, dinvf, zbias, zrows)

    (h2_3,) = _make_mlp(Np)(p0_st.reshape(2, Np, FH),
                            W1, b1.reshape(1, 2 * F), W2)

    p2_st = conv(h2_3.reshape(2 * Np, FH), srcf, dst2d, ews, dinvf, b2, zrows)

    out_st = _make_segmax(Np, G)(p2_st, batchp)

    out3 = out_st.reshape(2, G, FH)
    return jnp.concatenate([out3[0], out3[1]], axis=1)


# speedup vs baseline: 31.6911x; 1.0880x over previous
import functools

import jax
import jax.numpy as jnp
from jax import lax
from jax.experimental import pallas as pl
from jax.experimental.pallas import tpu as pltpu
from jax.experimental.pallas import tpu_sc as plsc

NC = 2
NS = 16
NW = NC * NS
L = 16

F = 64
FH = F // 2


def _mesh():
    return plsc.VectorSubcoreMesh(core_axis_name="c", subcore_axis_name="s")


def _sc_params():
    return pltpu.CompilerParams(
        needs_layout_passes=False, use_tc_tiling_on_sc=False)


def _vbcast(v, l):
    idx = jnp.full((L,), l, dtype=jnp.int32)
    dnums = lax.GatherDimensionNumbers(
        offset_dims=(), collapsed_slice_dims=(0,), start_index_map=(0,))
    return lax.gather(v, idx[:, None], dnums, (1,),
                      mode=lax.GatherScatterMode.PROMISE_IN_BOUNDS)


def _make_gather_deg(Np, Ep):
    n_w = Np // NW
    GCH = 112
    n_ch = n_w // GCH
    e_w = Ep // NW
    ECH = 1792
    e_ch = e_w // ECH

    @functools.partial(
        pl.kernel,
        out_type=(
            jax.ShapeDtypeStruct((2 * Np, FH), jnp.float32),
            jax.ShapeDtypeStruct((NW, Np), jnp.float32),
        ),
        mesh=_mesh(),
        compiler_params=_sc_params(),
        scratch_types=[
            pltpu.VMEM((n_w,), jnp.int32),
            pltpu.VMEM((GCH,), jnp.int32),
            pltpu.VMEM((GCH, FH), jnp.float32),
            pltpu.VMEM((Np,), jnp.float32),
            pltpu.VMEM((ECH,), jnp.int32),
            pltpu.VMEM((ECH,), jnp.float32),
            pltpu.SemaphoreType.DMA,
        ],
    )
    def k(embed2_h, xp_h, dstf_h, ewf_h, h0_h, degp_h,
          xb, idxb, rowsb, degb, dstb, ewb, sem):
        c = lax.axis_index("c")
        s = lax.axis_index("s")
        wid = s * NC + c
        nbase = wid * n_w

        pltpu.sync_copy(xp_h.at[pl.ds(nbase, n_w)], xb)

        def gather_half(off, out_base):
            def chunk(ch, _):
                def fill(i, _):
                    v = xb[pl.ds(ch * GCH + i * L, L)]
                    idxb[pl.ds(i * L, L)] = v * 2 + off
                    return 0
                lax.fori_loop(0, GCH // L, fill, 0)
                pltpu.async_copy(embed2_h.at[idxb], rowsb, sem).wait()
                pltpu.sync_copy(
                    rowsb, h0_h.at[pl.ds(out_base + ch * GCH, GCH)])
                return 0
            lax.fori_loop(0, n_ch, chunk, 0)

        gather_half(0, nbase)
        gather_half(1, Np + nbase)

        def zero(i, _):
            degb[pl.ds(i * L, L)] = jnp.zeros((L,), jnp.float32)
            return 0
        lax.fori_loop(0, Np // L, zero, 0)

        ebase = wid * e_w

        def echunk(ch, _):
            pltpu.sync_copy(dstf_h.at[pl.ds(ebase + ch * ECH, ECH)], dstb)
            pltpu.sync_copy(ewf_h.at[pl.ds(ebase + ch * ECH, ECH)], ewb)

            def grp(i, _):
                di = dstb[pl.ds(i * L, L)]
                wv = ewb[pl.ds(i * L, L)]
                plsc.addupdate_scatter(degb, [di], wv)
                return 0
            lax.fori_loop(0, ECH // L, grp, 0)
            return 0
        lax.fori_loop(0, e_ch, echunk, 0)

        pltpu.sync_copy(degb, degp_h.at[wid])

    return k


def _make_dinv(Np):
    R = Np // 128
    BR = 8
    grid = R // BR

    def body(degp_ref, dinv_ref):
        deg = jnp.sum(degp_ref[...], axis=0) + 1.0
        dinv_ref[...] = jnp.where(deg > 0, lax.rsqrt(deg), 0.0)

    return pl.pallas_call(
        body,
        grid=(grid,),
        in_specs=[pl.BlockSpec((NW, BR, 128), lambda i: (0, i, 0))],
        out_specs=[pl.BlockSpec((BR, 128), lambda i: (i, 0))],
        out_shape=[jax.ShapeDtypeStruct((R, 128), jnp.float32)],
    )


def _make_ews(Np, Ep):
    e_w = Ep // NW
    ECH = 1792
    n_ch = e_w // ECH

    @functools.partial(
        pl.kernel,
        out_type=jax.ShapeDtypeStruct((Ep,), jnp.float32),
        mesh=_mesh(),
        compiler_params=_sc_params(),
        scratch_types=[
            pltpu.VMEM((Np,), jnp.float32),
            pltpu.VMEM((ECH,), jnp.int32),
            pltpu.VMEM((ECH,), jnp.float32),
            pltpu.VMEM((ECH,), jnp.float32),
        ],
    )
    def k(dinv_h, srcf_h, ewf_h, ews_h, dv, srcb, ewb, ewsb):
        c = lax.axis_index("c")
        s = lax.axis_index("s")
        wid = s * NC + c
        ebase = wid * e_w
        pltpu.sync_copy(dinv_h, dv)

        def chunk(ch, _):
            base = ebase + ch * ECH
            pltpu.sync_copy(srcf_h.at[pl.ds(base, ECH)], srcb)
            pltpu.sync_copy(ewf_h.at[pl.ds(base, ECH)], ewb)

            def grp(i, _):
                sv = srcb[pl.ds(i * L, L)]
                ewsb[pl.ds(i * L, L)] = (ewb[pl.ds(i * L, L)]
                                         * plsc.load_gather(dv, [sv]))
                return 0
            lax.fori_loop(0, ECH // L, grp, 0)
            pltpu.sync_copy(ewsb, ews_h.at[pl.ds(base, ECH)])
            return 0
        lax.fori_loop(0, n_ch, chunk, 0)

    return k


def _make_conv(Np, Ep):
    e_t = Ep // NS
    ECH = 256
    n_sub = ECH // 128
    n_ch = e_t // ECH
    assert n_ch % 3 == 1 and n_ch >= 7
    r_t = Np // NS

    @functools.partial(
        pl.kernel,
        out_type=jax.ShapeDtypeStruct((2 * Np, FH), jnp.float32),
        mesh=_mesh(),
        compiler_params=_sc_params(),
        scratch_types=[
            pltpu.VMEM((3, ECH), jnp.int32),
            pltpu.VMEM((3, ECH), jnp.float32),
            pltpu.VMEM((3 * n_sub, 128), jnp.int32),
            pltpu.VMEM((3 * n_sub, 128), jnp.int32),
            pltpu.VMEM((3 * ECH, FH), jnp.float32),
            pltpu.VMEM((F,), jnp.float32),
            pltpu.VMEM_SHARED((Np, FH), jnp.float32),
            pltpu.SemaphoreType.DMA,
            pltpu.SemaphoreType.DMA,
            pltpu.SemaphoreType.DMA,
            pltpu.SemaphoreType.DMA,
            pltpu.SemaphoreType.DMA,
            pltpu.SemaphoreType.DMA,
            pltpu.SemaphoreType.DMA,
        ],
    )
    def k(g_h, srcf_h, dst2d_h, ewf_h, dinv_h, bias_h, zrows_h, out_h,
          srcb, ewb, gidx, didx, rowsb, biasb, acc,
          sg0, sg1, sg2, ss0, ss1, ss2, sem3):
        c = lax.axis_index("c")
        s = lax.axis_index("s")
        coff = c * Np
        semg = (sg0, sg1, sg2)
        sems = (ss0, ss1, ss2)

        pltpu.sync_copy(zrows_h, acc.at[pl.ds(s * r_t, r_t)])
        plsc.subcore_barrier()

        ebase = s * e_t

        def load_idx(ch, ph):
            base = ebase + ch * ECH
            pltpu.async_copy(srcf_h.at[pl.ds(base, ECH)], srcb.at[ph], sem3)
            pltpu.async_copy(ewf_h.at[pl.ds(base, ECH)], ewb.at[ph], sem3)
            pltpu.async_copy(dst2d_h.at[pl.ds(base // 128, n_sub)],
                             didx.at[pl.ds(ph * n_sub, n_sub)], sem3)

        def wait_idx(ch, ph):
            base = ebase + ch * ECH
            pltpu.make_async_copy(srcf_h.at[pl.ds(base, ECH)],
                                  srcb.at[ph], sem3).wait()
            pltpu.make_async_copy(ewf_h.at[pl.ds(base, ECH)],
                                  ewb.at[ph], sem3).wait()
            pltpu.make_async_copy(dst2d_h.at[pl.ds(base // 128, n_sub)],
                                  didx.at[pl.ds(ph * n_sub, n_sub)],
                                  sem3).wait()

        def fire_gather(ph):
            def fill(i, _):
                sv = srcb[ph, pl.ds(i * L, L)]
                j = i // 8
                lo = (i % 8) * L
                gidx[ph * n_sub + j, pl.ds(lo, L)] = sv + coff
                return 0
            lax.fori_loop(0, ECH // L, fill, 0)
            for j in range(n_sub):
                pltpu.async_copy(
                    g_h.at[gidx.at[ph * n_sub + j]],
                    rowsb.at[pl.ds((ph * n_sub + j) * 128, 128)], semg[ph])

        def wait_gather(ph):
            for j in range(n_sub):
                pltpu.make_async_copy(
                    g_h.at[gidx.at[ph * n_sub + j]],
                    rowsb.at[pl.ds((ph * n_sub + j) * 128, 128)],
                    semg[ph]).wait()

        def scale(ph):
            def grp(i, _):
                wv = ewb[ph, pl.ds(i * L, L)]
                for l in range(L):
                    nb = _vbcast(wv, l)
                    r = ph * ECH + i * L + l
                    rowsb[r, pl.ds(0, L)] = rowsb[r, pl.ds(0, L)] * nb
                    rowsb[r, pl.ds(L, L)] = rowsb[r, pl.ds(L, L)] * nb
                return 0
            lax.fori_loop(0, ECH // L, grp, 0)

        def fire_scatter(ph):
            for j in range(n_sub):
                pltpu.async_copy(
                    rowsb.at[pl.ds((ph * n_sub + j) * 128, 128)],
                    acc.at[didx.at[ph * n_sub + j]], sems[ph], add=True)

        def wait_scatter(ph):
            for j in range(n_sub):
                pltpu.make_async_copy(
                    rowsb.at[pl.ds((ph * n_sub + j) * 128, 128)],
                    acc.at[didx.at[ph * n_sub + j]], sems[ph]).wait()

        def body_sub(ch, ph, phn):
            wait_scatter(phn)
            load_idx(ch + 2, phn)
            wait_gather(ph)
            scale(ph)
            fire_scatter(ph)
            wait_idx(ch + 2, phn)
            fire_gather(phn)

        load_idx(0, 0)
        wait_idx(0, 0)
        fire_gather(0)
        load_idx(1, 1)
        wait_idx(1, 1)
        fire_gather(1)
        load_idx(2, 2)
        wait_gather(0)
        scale(0)
        fire_scatter(0)
        wait_idx(2, 2)
        fire_gather(2)

        def pbody(p, _):
            ch = 3 * p
            body_sub(ch + 1, 1, 0)
            body_sub(ch + 2, 2, 1)
            body_sub(ch + 3, 0, 2)
            return 0
        lax.fori_loop(0, (n_ch - 4) // 3, pbody, 0)

        body_sub(n_ch - 3, 1, 0)
        wait_scatter(1)
        wait_gather(2)
        scale(2)
        fire_scatter(2)
        wait_scatter(2)
        wait_gather(0)
        scale(0)
        fire_scatter(0)
        wait_scatter(0)

        plsc.subcore_barrier()

        pltpu.sync_copy(bias_h, biasb)
        b_lo = c * FH
        WCH = 224
        def wout(i, _):
            r0 = s * r_t + i * WCH
            pltpu.sync_copy(acc.at[pl.ds(r0, WCH)],
                            rowsb.at[pl.ds(0, WCH)])
            pltpu.sync_copy(g_h.at[pl.ds(coff + r0, WCH)],
                            rowsb.at[pl.ds(256, WCH)])
            pltpu.sync_copy(dinv_h.at[pl.ds(r0, WCH)],
                            ewb.at[0, pl.ds(0, WCH)])

            bv0 = biasb[pl.ds(b_lo, L)]
            bv1 = biasb[pl.ds(b_lo + L, L)]

            def wgrp(i2, _):
                dv = ewb[0, pl.ds(i2 * L, L)]
                for l in range(L):
                    db = _vbcast(dv, l)
                    d2b = db * db
                    r = i2 * L + l
                    rowsb[r, pl.ds(0, L)] = (
                        rowsb[r, pl.ds(0, L)] * db
                        + rowsb[256 + r, pl.ds(0, L)] * d2b + bv0)
                    rowsb[r, pl.ds(L, L)] = (
                        rowsb[r, pl.ds(L, L)] * db
                        + rowsb[256 + r, pl.ds(L, L)] * d2b + bv1)
                return 0
            lax.fori_loop(0, WCH // L, wgrp, 0)

            pltpu.sync_copy(rowsb.at[pl.ds(0, WCH)],
                            out_h.at[pl.ds(coff + r0, WCH)])
            return 0
        lax.fori_loop(0, r_t // WCH, wout, 0)

    return k


def _make_mlp(Np, BR=1024):
    grid = Np // BR

    HV = Np * FH // 128
    BV = 256
    grid = HV // BV

    def body(p_ref, w1a_ref, w1b_ref, b1_ref, w2p_ref, h2_ref):
        hp = lax.Precision.HIGHEST
        h = (jnp.dot(p_ref[0], w1a_ref[...], precision=hp,
                     preferred_element_type=jnp.float32)
             + jnp.dot(p_ref[1], w1b_ref[...], precision=hp,
                       preferred_element_type=jnp.float32)
             + b1_ref[...])
        h = jnp.maximum(h, 0.0)
        h2 = jnp.dot(h, w2p_ref[...], precision=hp,
                     preferred_element_type=jnp.float32)
        h2_ref[0] = h2[:, :128]
        h2_ref[1] = h2[:, 128:]

    return pl.pallas_call(
        body,
        grid=(grid,),
        in_specs=[pl.BlockSpec((2, BV, 128), lambda i: (0, i, 0)),
                  pl.BlockSpec((128, 512), lambda i: (0, 0)),
                  pl.BlockSpec((128, 512), lambda i: (0, 0)),
                  pl.BlockSpec((1, 512), lambda i: (0, 0)),
                  pl.BlockSpec((512, 256), lambda i: (0, 0))],
        out_specs=[pl.BlockSpec((2, BV, 128), lambda i: (0, i, 0))],
        out_shape=[jax.ShapeDtypeStruct((2, HV, 128), jnp.float32)],
    )


def _make_segmax(Np, G):
    r_t = Np // NS
    TG = G + 8

    @functools.partial(
        pl.kernel,
        out_type=jax.ShapeDtypeStruct((2 * G, FH), jnp.float32),
        mesh=_mesh(),
        compiler_params=_sc_params(),
        scratch_types=[
            pltpu.VMEM((r_t, FH), jnp.float32),
            pltpu.VMEM((r_t,), jnp.int32),
            pltpu.VMEM((TG, FH), jnp.float32),
            pltpu.VMEM((8, FH), jnp.float32),
            pltpu.VMEM((8, FH), jnp.float32),
            pltpu.VMEM_SHARED((NS, G, FH), jnp.float32),
            pltpu.SemaphoreType.DMA,
        ],
    )
    def k(p_h, batch_h, out_h, rowsb, batchb, tb, racc, rtmp, shared, sem):
        c = lax.axis_index("c")
        s = lax.axis_index("s")
        coff = c * Np

        pltpu.sync_copy(p_h.at[pl.ds(coff + s * r_t, r_t)], rowsb)
        pltpu.sync_copy(batch_h.at[pl.ds(s * r_t, r_t)], batchb)

        ninf = jnp.full((L,), -jnp.inf, jnp.float32)

        def zt(i, _):
            tb[i, pl.ds(0, L)] = ninf
            tb[i, pl.ds(L, L)] = ninf
            return 0
        lax.fori_loop(0, TG, zt, 0)

        iota = lax.iota(jnp.int32, L)

        def grp(i, _):
            bv = batchb[pl.ds(i * L, L)]
            for l in range(L):
                gb = _vbcast(bv, l)
                r = i * L + l
                r0 = rowsb[r, pl.ds(0, L)]
                r1 = rowsb[r, pl.ds(L, L)]
                cur0 = plsc.load_gather(tb, [gb, iota])
                cur1 = plsc.load_gather(tb, [gb, iota + L])
                plsc.store_scatter(tb, [gb, iota], jnp.maximum(cur0, r0))
                plsc.store_scatter(tb, [gb, iota + L], jnp.maximum(cur1, r1))
            return 0
        lax.fori_loop(0, r_t // L, grp, 0)

        pltpu.sync_copy(tb.at[pl.ds(0, G)], shared.at[s])
        plsc.subcore_barrier()

        gs = G // NS
        pltpu.sync_copy(shared.at[0, pl.ds(s * gs, gs)], racc)

        for t in range(1, NS):
            pltpu.sync_copy(shared.at[t, pl.ds(s * gs, gs)], rtmp)
            for r in range(gs):
                for j in (0, L):
                    racc[r, pl.ds(j, L)] = jnp.maximum(
                        racc[r, pl.ds(j, L)], rtmp[r, pl.ds(j, L)])

        pltpu.sync_copy(racc, out_h.at[pl.ds(c * G + s * gs, gs)])

    return k


def kernel(x, edge_index, edge_attr, batch, embed, W1, b1, W2, b2):
    N = x.shape[0]
    E = edge_index.shape[1]
    V = embed.shape[0]
    G = 128

    Np = 50176
    Ep = 802816
    assert N <= Np and E <= Ep

    pad_n = Np - N
    pad_e = Ep - E

    xp = jnp.concatenate([x, (jnp.arange(pad_n, dtype=jnp.int32) % V)])
    srcf = jnp.concatenate(
        [edge_index[0], (jnp.arange(pad_e, dtype=jnp.int32) * 131) % N])
    dstf = jnp.concatenate(
        [edge_index[1], (jnp.arange(pad_e, dtype=jnp.int32) * 137) % N])
    ewf = jnp.concatenate([edge_attr, jnp.zeros((pad_e,), jnp.float32)])
    batchp = jnp.concatenate([batch, jnp.full((pad_n,), G, jnp.int32)])
    dst2d = dstf.reshape(Ep // 128, 128)

    embed2 = embed.reshape(V * 2, FH)

    h0_st, degp = _make_gather_deg(Np, Ep)(embed2, xp, dstf, ewf)

    (dinv2d,) = _make_dinv(Np)(degp.reshape(NW, Np // 128, 128))
    dinvf = dinv2d.reshape(Np)

    ews = _make_ews(Np, Ep)(dinvf, srcf, ewf)

    conv = _make_conv(Np, Ep)
    zbias = jnp.zeros((F,), jnp.float32)
    zrows = jnp.zeros((Np // NS, FH), jnp.float32)

    p0_st = conv(h0_st, srcf, dst2d, ews, dinvf, zbias, zrows)

    eye4 = jnp.eye(4, dtype=jnp.float32)
    w1a = jnp.kron(eye4, W1[:FH])
    w1b = jnp.kron(eye4, W1[FH:])
    b1big = jnp.tile(b1, 4).reshape(1, 512)
    w2p = jnp.concatenate([jnp.kron(eye4, W2[:, :FH]),
                           jnp.kron(eye4, W2[:, FH:])], axis=1)
    HV = Np * FH // 128
    (h2_v,) = _make_mlp(Np)(p0_st.reshape(2, HV, 128), w1a, w1b, b1big, w2p)

    p2_st = conv(h2_v.reshape(2 * Np, FH), srcf, dst2d, ews, dinvf, b2, zrows)

    out_st = _make_segmax(Np, G)(p2_st, batchp)

    out3 = out_st.reshape(2, G, FH)
    return jnp.concatenate([out3[0], out3[1]], axis=1)
